# Initial kernel scaffold; baseline (speedup 1.0000x reference)
import jax
import jax.numpy as jnp
from jax.experimental import pallas as pl


def kernel(arm_cls, arm_loc, odm_cls, odm_loc, anchors):
    raise NotImplementedError("write your pallas kernel here")


# speedup vs baseline: 9.1275x; 9.1275x over previous
import jax
import jax.numpy as jnp
from jax.experimental import pallas as pl
from jax.experimental.pallas import tpu as pltpu

N = 20000
NCLS = 20
MAXP = 200
CONF_T = 0.01
ART = 0.99
NMST = 0.45
VAR = (0.1, 0.1, 0.2, 0.2)


def _nms_body(arm_bg_ref, arm_loc_ref, odm_cls_ref, odm_loc_ref, anchors_ref,
              out_ref, scores_ref, boxes_ref):
    a_cx = anchors_ref[0:1, :]
    a_cy = anchors_ref[1:2, :]
    a_w = anchors_ref[2:3, :]
    a_h = anchors_ref[3:4, :]

    r_cx = arm_loc_ref[0:1, :] * VAR[0] * a_w + a_cx
    r_cy = arm_loc_ref[1:2, :] * VAR[1] * a_h + a_cy
    r_w = a_w * jnp.exp(arm_loc_ref[2:3, :] * VAR[2])
    r_h = a_h * jnp.exp(arm_loc_ref[3:4, :] * VAR[3])

    d_cx = odm_loc_ref[0:1, :] * VAR[0] * r_w + r_cx
    d_cy = odm_loc_ref[1:2, :] * VAR[1] * r_h + r_cy
    d_w = r_w * jnp.exp(odm_loc_ref[2:3, :] * VAR[2])
    d_h = r_h * jnp.exp(odm_loc_ref[3:4, :] * VAR[3])

    xmin = jnp.clip(d_cx - d_w * 0.5, 0.0, 1.0)
    ymin = jnp.clip(d_cy - d_h * 0.5, 0.0, 1.0)
    xmax = jnp.clip(d_cx + d_w * 0.5, 0.0, 1.0)
    ymax = jnp.clip(d_cy + d_h * 0.5, 0.0, 1.0)
    area2 = (jnp.maximum(xmax - xmin, 0.0) * jnp.maximum(ymax - ymin, 0.0))

    boxes_ref[0:1, :] = xmin
    boxes_ref[1:2, :] = ymin
    boxes_ref[2:3, :] = xmax
    boxes_ref[3:4, :] = ymax
    boxes_ref[4:5, :] = area2

    keep = 1.0 - (arm_bg_ref[0:1, :] >= ART).astype(jnp.float32)
    cls = odm_cls_ref[:, :] * keep
    scores_ref[:, :] = jnp.where(cls > CONF_T, cls, 0.0)

    iota = jax.lax.broadcasted_iota(jnp.int32, (NCLS, N), 1)

    def step(i, _):
        scores = scores_ref[:, :]
        xmin = boxes_ref[0:1, :]
        ymin = boxes_ref[1:2, :]
        xmax = boxes_ref[2:3, :]
        ymax = boxes_ref[3:4, :]
        area2 = boxes_ref[4:5, :]

        m = jnp.max(scores, axis=1, keepdims=True)
        idx = jnp.min(jnp.where(scores == m, iota, N), axis=1,
                      keepdims=True)
        onehot = (iota == idx).astype(jnp.float32)
        b_xmin = jnp.sum(onehot * xmin, axis=1, keepdims=True)
        b_ymin = jnp.sum(onehot * ymin, axis=1, keepdims=True)
        b_xmax = jnp.sum(onehot * xmax, axis=1, keepdims=True)
        b_ymax = jnp.sum(onehot * ymax, axis=1, keepdims=True)
        valid = m > 0.0

        ixmin = jnp.maximum(b_xmin, xmin)
        iymin = jnp.maximum(b_ymin, ymin)
        ixmax = jnp.minimum(b_xmax, xmax)
        iymax = jnp.minimum(b_ymax, ymax)
        iw = jnp.maximum(ixmax - ixmin, 0.0)
        ih = jnp.maximum(iymax - iymin, 0.0)
        inter = iw * ih
        area1 = (jnp.maximum(b_xmax - b_xmin, 0.0)
                 * jnp.maximum(b_ymax - b_ymin, 0.0))
        union = area1 + area2 - inter
        iou = inter / jnp.maximum(union, 1e-8)
        scores_ref[:, :] = jnp.where(iou >= NMST, 0.0, scores)

        zero = jnp.zeros_like(m)
        row = jnp.concatenate(
            [jnp.where(valid, b_xmin, zero),
             jnp.where(valid, b_ymin, zero),
             jnp.where(valid, b_xmax, zero),
             jnp.where(valid, b_ymax, zero),
             jnp.where(valid, m, zero)], axis=1)
        out_ref[pl.ds(i, 1)] = row[None]
        return 0

    jax.lax.fori_loop(0, MAXP, step, 0)


def kernel(arm_cls, arm_loc, odm_cls, odm_loc, anchors):
    arm_bg = arm_cls[0, :, 0][None, :]
    arm_loc_t = arm_loc[0].T
    odm_cls_t = odm_cls[0].T[1:NCLS + 1]
    odm_loc_t = odm_loc[0].T
    anchors_t = anchors.T

    out = pl.pallas_call(
        _nms_body,
        out_shape=jax.ShapeDtypeStruct((MAXP, NCLS, 5), jnp.float32),
        scratch_shapes=[
            pltpu.VMEM((NCLS, N), jnp.float32),
            pltpu.VMEM((8, N), jnp.float32),
        ],
    )(arm_bg, arm_loc_t, odm_cls_t, odm_loc_t, anchors_t)

    return jnp.transpose(out, (1, 0, 2))[None]


# speedup vs baseline: 16.1877x; 1.7735x over previous
import functools

import jax
import jax.numpy as jnp
from jax import lax
from jax.experimental import pallas as pl
from jax.experimental.pallas import tpu as pltpu
from jax.experimental.pallas import tpu_sc as plsc

N = 20000
NCLS = 20
MAXP = 200
PADP = 256
CONF_T = 0.01
ART = 0.99
NMST = 0.45
VAR = (0.1, 0.1, 0.2, 0.2)
K = 2048
TAU = 0.93


def _decode_body(arm_bg_ref, arm_loc_ref, odm_cls_ref, odm_loc_ref,
                 anchors_ref, scores_ref, boxes_ref):
    a_cx = anchors_ref[0:1, :]
    a_cy = anchors_ref[1:2, :]
    a_w = anchors_ref[2:3, :]
    a_h = anchors_ref[3:4, :]

    r_cx = arm_loc_ref[0:1, :] * VAR[0] * a_w + a_cx
    r_cy = arm_loc_ref[1:2, :] * VAR[1] * a_h + a_cy
    r_w = a_w * jnp.exp(arm_loc_ref[2:3, :] * VAR[2])
    r_h = a_h * jnp.exp(arm_loc_ref[3:4, :] * VAR[3])

    d_cx = odm_loc_ref[0:1, :] * VAR[0] * r_w + r_cx
    d_cy = odm_loc_ref[1:2, :] * VAR[1] * r_h + r_cy
    d_w = r_w * jnp.exp(odm_loc_ref[2:3, :] * VAR[2])
    d_h = r_h * jnp.exp(odm_loc_ref[3:4, :] * VAR[3])

    xmin = jnp.clip(d_cx - d_w * 0.5, 0.0, 1.0)
    ymin = jnp.clip(d_cy - d_h * 0.5, 0.0, 1.0)
    xmax = jnp.clip(d_cx + d_w * 0.5, 0.0, 1.0)
    ymax = jnp.clip(d_cy + d_h * 0.5, 0.0, 1.0)

    boxes_ref[0:1, :] = xmin
    boxes_ref[1:2, :] = ymin
    boxes_ref[2:3, :] = xmax
    boxes_ref[3:4, :] = ymax
    z = jnp.zeros_like(xmin)
    boxes_ref[4:5, :] = z
    boxes_ref[5:6, :] = z
    boxes_ref[6:7, :] = z
    boxes_ref[7:8, :] = z

    keep = 1.0 - (arm_bg_ref[0:1, :] >= ART).astype(jnp.float32)
    cls = odm_cls_ref[:, :] * keep
    scores_ref[:, :] = jnp.where(cls > CONF_T, cls, 0.0)


_sc_mesh = plsc.VectorSubcoreMesh(core_axis_name="c", subcore_axis_name="s")


@functools.partial(
    pl.kernel,
    mesh=_sc_mesh,
    compiler_params=pltpu.CompilerParams(needs_layout_passes=False),
    out_type=(
        jax.ShapeDtypeStruct((NCLS, K), jnp.float32),
        jax.ShapeDtypeStruct((NCLS, K), jnp.float32),
        jax.ShapeDtypeStruct((NCLS, K), jnp.float32),
        jax.ShapeDtypeStruct((NCLS, K), jnp.float32),
        jax.ShapeDtypeStruct((NCLS, K), jnp.float32),
    ),
    scratch_types=[
        pltpu.VMEM((N,), jnp.float32),
        pltpu.VMEM((N,), jnp.float32),
        pltpu.VMEM((N,), jnp.float32),
        pltpu.VMEM((N,), jnp.float32),
        pltpu.VMEM((N,), jnp.float32),
        pltpu.VMEM((K,), jnp.float32),
        pltpu.VMEM((K,), jnp.int32),
        pltpu.VMEM((K,), jnp.float32),
    ],
)
def _sc_compact(scores_hbm, boxes_hbm,
                cs_hbm, cx_hbm, cy_hbm, cxx_hbm, cyy_hbm,
                s_v, x0_v, y0_v, x1_v, y1_v, cs_v, ci_v, plane_v):
    wid = lax.axis_index("c") * 16 + lax.axis_index("s")

    @pl.when(wid < NCLS)
    def _():
        pltpu.sync_copy(scores_hbm.at[wid], s_v)
        pltpu.sync_copy(boxes_hbm.at[0], x0_v)
        pltpu.sync_copy(boxes_hbm.at[1], y0_v)
        pltpu.sync_copy(boxes_hbm.at[2], x1_v)
        pltpu.sync_copy(boxes_hbm.at[3], y1_v)

        zf = jnp.zeros((16,), jnp.float32)
        zi = jnp.zeros((16,), jnp.int32)

        def zero_loop(j, _):
            cs_v[pl.ds(j * 16, 16)] = zf
            ci_v[pl.ds(j * 16, 16)] = zi
            return 0

        lax.fori_loop(0, K // 16, zero_loop, 0)

        lanes = lax.iota(jnp.int32, 16)

        def scan(i, off):
            v = s_v[pl.ds(i * 16, 16)]
            msk = v > TAU
            cnt = jnp.sum(jnp.where(msk, jnp.ones((16,), jnp.int32),
                                    jnp.zeros((16,), jnp.int32)))

            @pl.when(off <= K - 16)
            def _():
                plsc.store_compressed(cs_v.at[pl.ds(off, 16)], v, mask=msk)
                plsc.store_compressed(ci_v.at[pl.ds(off, 16)],
                                      lanes + i * 16, mask=msk)

            return off + cnt

        off_final = lax.fori_loop(0, N // 16, scan, 0)

        @pl.when(off_final > K - 16)
        def _():
            lax.fori_loop(0, K // 16, zero_loop, 0)

        pltpu.sync_copy(cs_v, cs_hbm.at[wid])

        for coord_v, out_hbm in ((x0_v, cx_hbm), (y0_v, cy_hbm),
                                 (x1_v, cxx_hbm), (y1_v, cyy_hbm)):
            def extract(j, _, coord_v=coord_v):
                idxv = ci_v[pl.ds(j * 16, 16)]
                plane_v[pl.ds(j * 16, 16)] = plsc.load_gather(coord_v, [idxv])
                return 0

            lax.fori_loop(0, K // 16, extract, 0)
            pltpu.sync_copy(plane_v, out_hbm.at[wid])


def _nms_body(cs_ref, cx_ref, cy_ref, cxx_ref, cyy_ref, scores_in_ref,
              boxes_ref, out_ref, csc_ref, sfull_ref):
    csc_ref[:, :] = cs_ref[:, :]
    out_ref[:, :, :] = jnp.zeros((5, NCLS, PADP), jnp.float32)

    iota_k = lax.broadcasted_iota(jnp.int32, (NCLS, K), 1)
    lane = lax.broadcasted_iota(jnp.int32, (NCLS, PADP), 1)

    def cond(carry):
        return carry[0]

    def body(carry):
        _, count, r = carry
        csc = csc_ref[:, :]
        m = jnp.max(csc, axis=1, keepdims=True)
        idx = jnp.min(jnp.where(csc == m, iota_k, K), axis=1,
                      keepdims=True)
        onehot = iota_k == idx
        b_xmin = jnp.sum(jnp.where(onehot, cx_ref[:, :], 0.0), axis=1,
                         keepdims=True)
        b_ymin = jnp.sum(jnp.where(onehot, cy_ref[:, :], 0.0), axis=1,
                         keepdims=True)
        b_xmax = jnp.sum(jnp.where(onehot, cxx_ref[:, :], 0.0), axis=1,
                         keepdims=True)
        b_ymax = jnp.sum(jnp.where(onehot, cyy_ref[:, :], 0.0), axis=1,
                         keepdims=True)
        valid = (m > 0.0) & (count < MAXP)

        sx0 = out_ref[0, :, :]
        sy0 = out_ref[1, :, :]
        sx1 = out_ref[2, :, :]
        sy1 = out_ref[3, :, :]
        ixmin = jnp.maximum(sx0, b_xmin)
        iymin = jnp.maximum(sy0, b_ymin)
        ixmax = jnp.minimum(sx1, b_xmax)
        iymax = jnp.minimum(sy1, b_ymax)
        iw = jnp.maximum(ixmax - ixmin, 0.0)
        ih = jnp.maximum(iymax - iymin, 0.0)
        inter = iw * ih
        area1 = (jnp.maximum(sx1 - sx0, 0.0)
                 * jnp.maximum(sy1 - sy0, 0.0))
        area2 = (jnp.maximum(b_xmax - b_xmin, 0.0)
                 * jnp.maximum(b_ymax - b_ymin, 0.0))
        union = area1 + area2 - inter
        iou = inter / jnp.maximum(union, 1e-8)
        supp = jnp.any(iou >= NMST, axis=1, keepdims=True)

        acc = valid & jnp.logical_not(supp)
        sel = (lane == count) & acc
        out_ref[0, :, :] = jnp.where(sel, b_xmin, out_ref[0, :, :])
        out_ref[1, :, :] = jnp.where(sel, b_ymin, out_ref[1, :, :])
        out_ref[2, :, :] = jnp.where(sel, b_xmax, out_ref[2, :, :])
        out_ref[3, :, :] = jnp.where(sel, b_ymax, out_ref[3, :, :])
        out_ref[4, :, :] = jnp.where(sel, m, out_ref[4, :, :])
        count = count + acc.astype(jnp.int32)
        csc_ref[:, :] = jnp.where(onehot, 0.0, csc)

        cont = jnp.any((count < MAXP) & (m > 0.0)) & (r < K + MAXP + 8)
        return cont, count, r + 1

    count0 = jnp.zeros((NCLS, 1), jnp.int32)
    _, count, _ = lax.while_loop(cond, body, (True, count0, 0))

    flags = count < MAXP

    @pl.when(jnp.any(flags))
    def _():
        sfull_ref[:, :] = jnp.where(flags, scores_in_ref[:, :], 0.0)
        flag_l = jnp.broadcast_to(flags, (NCLS, PADP))
        out_ref[0, :, :] = jnp.where(flag_l, 0.0, out_ref[0, :, :])
        out_ref[1, :, :] = jnp.where(flag_l, 0.0, out_ref[1, :, :])
        out_ref[2, :, :] = jnp.where(flag_l, 0.0, out_ref[2, :, :])
        out_ref[3, :, :] = jnp.where(flag_l, 0.0, out_ref[3, :, :])
        out_ref[4, :, :] = jnp.where(flag_l, 0.0, out_ref[4, :, :])

        iota_n = lax.broadcasted_iota(jnp.int32, (NCLS, N), 1)

        def step(i, _):
            scores = sfull_ref[:, :]
            xmin = boxes_ref[0:1, :]
            ymin = boxes_ref[1:2, :]
            xmax = boxes_ref[2:3, :]
            ymax = boxes_ref[3:4, :]

            m = jnp.max(scores, axis=1, keepdims=True)
            idx = jnp.min(jnp.where(scores == m, iota_n, N), axis=1,
                          keepdims=True)
            onehot = (iota_n == idx).astype(jnp.float32)
            b_xmin = jnp.sum(onehot * xmin, axis=1, keepdims=True)
            b_ymin = jnp.sum(onehot * ymin, axis=1, keepdims=True)
            b_xmax = jnp.sum(onehot * xmax, axis=1, keepdims=True)
            b_ymax = jnp.sum(onehot * ymax, axis=1, keepdims=True)
            valid = m > 0.0

            ixmin = jnp.maximum(b_xmin, xmin)
            iymin = jnp.maximum(b_ymin, ymin)
            ixmax = jnp.minimum(b_xmax, xmax)
            iymax = jnp.minimum(b_ymax, ymax)
            iw = jnp.maximum(ixmax - ixmin, 0.0)
            ih = jnp.maximum(iymax - iymin, 0.0)
            inter = iw * ih
            area1 = (jnp.maximum(b_xmax - b_xmin, 0.0)
                     * jnp.maximum(b_ymax - b_ymin, 0.0))
            area2 = (jnp.maximum(xmax - xmin, 0.0)
                     * jnp.maximum(ymax - ymin, 0.0))
            union = area1 + area2 - inter
            iou = inter / jnp.maximum(union, 1e-8)
            sfull_ref[:, :] = jnp.where(iou >= NMST, 0.0, scores)

            sel = (lane == i) & valid
            out_ref[0, :, :] = jnp.where(sel, b_xmin, out_ref[0, :, :])
            out_ref[1, :, :] = jnp.where(sel, b_ymin, out_ref[1, :, :])
            out_ref[2, :, :] = jnp.where(sel, b_xmax, out_ref[2, :, :])
            out_ref[3, :, :] = jnp.where(sel, b_ymax, out_ref[3, :, :])
            out_ref[4, :, :] = jnp.where(sel, m, out_ref[4, :, :])
            return 0

        lax.fori_loop(0, MAXP, step, 0)


def kernel(arm_cls, arm_loc, odm_cls, odm_loc, anchors):
    arm_bg = arm_cls[0, :, 0][None, :]
    arm_loc_t = arm_loc[0].T
    odm_cls_t = odm_cls[0].T[1:NCLS + 1]
    odm_loc_t = odm_loc[0].T
    anchors_t = anchors.T

    scores, boxes8 = pl.pallas_call(
        _decode_body,
        out_shape=(
            jax.ShapeDtypeStruct((NCLS, N), jnp.float32),
            jax.ShapeDtypeStruct((8, N), jnp.float32),
        ),
    )(arm_bg, arm_loc_t, odm_cls_t, odm_loc_t, anchors_t)

    cs, cx, cy, cxx, cyy = _sc_compact(scores, boxes8)

    out = pl.pallas_call(
        _nms_body,
        out_shape=jax.ShapeDtypeStruct((5, NCLS, PADP), jnp.float32),
        scratch_shapes=[
            pltpu.VMEM((NCLS, K), jnp.float32),
            pltpu.VMEM((NCLS, N), jnp.float32),
        ],
    )(cs, cx, cy, cxx, cyy, scores, boxes8)

    return jnp.transpose(out[:, :, :MAXP], (1, 2, 0))[None]


# speedup vs baseline: 43.8244x; 2.7073x over previous
import functools

import jax
import jax.numpy as jnp
from jax import lax
from jax.experimental import pallas as pl
from jax.experimental.pallas import tpu as pltpu
from jax.experimental.pallas import tpu_sc as plsc

N = 20000
NCLS = 20
MAXP = 200
PADP = 256
CONF_T = 0.01
ART = 0.99
NMST = 0.45
VAR = (0.1, 0.1, 0.2, 0.2)
K = 1280
TAU = 0.95


def _decode_body(arm_bg_ref, arm_loc_ref, odm_cls_ref, odm_loc_ref,
                 anchors_ref, scores_ref, boxes_ref):
    a_cx = anchors_ref[0:1, :]
    a_cy = anchors_ref[1:2, :]
    a_w = anchors_ref[2:3, :]
    a_h = anchors_ref[3:4, :]

    r_cx = arm_loc_ref[0:1, :] * VAR[0] * a_w + a_cx
    r_cy = arm_loc_ref[1:2, :] * VAR[1] * a_h + a_cy
    r_w = a_w * jnp.exp(arm_loc_ref[2:3, :] * VAR[2])
    r_h = a_h * jnp.exp(arm_loc_ref[3:4, :] * VAR[3])

    d_cx = odm_loc_ref[0:1, :] * VAR[0] * r_w + r_cx
    d_cy = odm_loc_ref[1:2, :] * VAR[1] * r_h + r_cy
    d_w = r_w * jnp.exp(odm_loc_ref[2:3, :] * VAR[2])
    d_h = r_h * jnp.exp(odm_loc_ref[3:4, :] * VAR[3])

    xmin = jnp.clip(d_cx - d_w * 0.5, 0.0, 1.0)
    ymin = jnp.clip(d_cy - d_h * 0.5, 0.0, 1.0)
    xmax = jnp.clip(d_cx + d_w * 0.5, 0.0, 1.0)
    ymax = jnp.clip(d_cy + d_h * 0.5, 0.0, 1.0)

    boxes_ref[0:1, :] = xmin
    boxes_ref[1:2, :] = ymin
    boxes_ref[2:3, :] = xmax
    boxes_ref[3:4, :] = ymax
    z = jnp.zeros_like(xmin)
    boxes_ref[4:5, :] = z
    boxes_ref[5:6, :] = z
    boxes_ref[6:7, :] = z
    boxes_ref[7:8, :] = z

    keep = 1.0 - (arm_bg_ref[0:1, :] >= ART).astype(jnp.float32)
    cls = odm_cls_ref[:, :] * keep
    scores_ref[:, :] = jnp.where(cls > CONF_T, cls, 0.0)


_sc_mesh = plsc.VectorSubcoreMesh(core_axis_name="c", subcore_axis_name="s")


@functools.partial(
    pl.kernel,
    mesh=_sc_mesh,
    compiler_params=pltpu.CompilerParams(needs_layout_passes=False),
    out_type=(
        jax.ShapeDtypeStruct((NCLS, K), jnp.float32),
        jax.ShapeDtypeStruct((NCLS, K), jnp.float32),
        jax.ShapeDtypeStruct((NCLS, K), jnp.float32),
        jax.ShapeDtypeStruct((NCLS, K), jnp.float32),
        jax.ShapeDtypeStruct((NCLS, K), jnp.float32),
    ),
    scratch_types=[
        pltpu.VMEM((N,), jnp.float32),
        pltpu.VMEM((N,), jnp.float32),
        pltpu.VMEM((N,), jnp.float32),
        pltpu.VMEM((N,), jnp.float32),
        pltpu.VMEM((N,), jnp.float32),
        pltpu.VMEM((K,), jnp.float32),
        pltpu.VMEM((K,), jnp.int32),
        pltpu.VMEM((K,), jnp.float32),
    ],
)
def _sc_compact(scores_hbm, boxes_hbm,
                cs_hbm, cx_hbm, cy_hbm, cxx_hbm, cyy_hbm,
                s_v, x0_v, y0_v, x1_v, y1_v, cs_v, ci_v, plane_v):
    wid = lax.axis_index("c") * 16 + lax.axis_index("s")

    @pl.when(wid < NCLS)
    def _():
        pltpu.sync_copy(scores_hbm.at[wid], s_v)
        pltpu.sync_copy(boxes_hbm.at[0], x0_v)
        pltpu.sync_copy(boxes_hbm.at[1], y0_v)
        pltpu.sync_copy(boxes_hbm.at[2], x1_v)
        pltpu.sync_copy(boxes_hbm.at[3], y1_v)

        zf = jnp.zeros((16,), jnp.float32)
        zi = jnp.zeros((16,), jnp.int32)

        def zero_loop(j, _):
            cs_v[pl.ds(j * 16, 16)] = zf
            ci_v[pl.ds(j * 16, 16)] = zi
            return 0

        lax.fori_loop(0, K // 16, zero_loop, 0)

        lanes = lax.iota(jnp.int32, 16)

        def scan(i, off):
            v = s_v[pl.ds(i * 16, 16)]
            msk = v > TAU
            cnt = jnp.sum(jnp.where(msk, jnp.ones((16,), jnp.int32),
                                    jnp.zeros((16,), jnp.int32)))

            @pl.when(off <= K - 16)
            def _():
                plsc.store_compressed(cs_v.at[pl.ds(off, 16)], v, mask=msk)
                plsc.store_compressed(ci_v.at[pl.ds(off, 16)],
                                      lanes + i * 16, mask=msk)

            return off + cnt

        off_final = lax.fori_loop(0, N // 16, scan, 0)

        @pl.when(off_final > K - 16)
        def _():
            lax.fori_loop(0, K // 16, zero_loop, 0)

        pltpu.sync_copy(cs_v, cs_hbm.at[wid])

        for coord_v, out_hbm in ((x0_v, cx_hbm), (y0_v, cy_hbm),
                                 (x1_v, cxx_hbm), (y1_v, cyy_hbm)):
            def extract(j, _, coord_v=coord_v):
                idxv = ci_v[pl.ds(j * 16, 16)]
                plane_v[pl.ds(j * 16, 16)] = plsc.load_gather(coord_v, [idxv])
                return 0

            lax.fori_loop(0, K // 16, extract, 0)
            pltpu.sync_copy(plane_v, out_hbm.at[wid])


def _nms_body(cs_ref, cx_ref, cy_ref, cxx_ref, cyy_ref, scores_in_ref,
              boxes_ref, out_ref, csc_ref, carea_ref, sfull_ref):
    csc_ref[:, :] = cs_ref[:, :]
    carea_ref[:, :] = (jnp.maximum(cxx_ref[:, :] - cx_ref[:, :], 0.0)
                       * jnp.maximum(cyy_ref[:, :] - cy_ref[:, :], 0.0))
    out_ref[:, :, :] = jnp.zeros((5, NCLS, PADP), jnp.float32)

    iota_k = lax.broadcasted_iota(jnp.int32, (NCLS, K), 1)
    lane = lax.broadcasted_iota(jnp.int32, (NCLS, PADP), 1)

    def body(i, count):
        csc = csc_ref[:, :]
        cx = cx_ref[:, :]
        cy = cy_ref[:, :]
        cxx = cxx_ref[:, :]
        cyy = cyy_ref[:, :]
        m = jnp.max(csc, axis=1, keepdims=True)
        idx = jnp.min(jnp.where(csc == m, iota_k, K), axis=1,
                      keepdims=True)
        onehot = iota_k == idx
        b_xmin = jnp.sum(jnp.where(onehot, cx, 0.0), axis=1,
                         keepdims=True)
        b_ymin = jnp.sum(jnp.where(onehot, cy, 0.0), axis=1,
                         keepdims=True)
        b_xmax = jnp.sum(jnp.where(onehot, cxx, 0.0), axis=1,
                         keepdims=True)
        b_ymax = jnp.sum(jnp.where(onehot, cyy, 0.0), axis=1,
                         keepdims=True)
        valid = m > 0.0

        ixmin = jnp.maximum(b_xmin, cx)
        iymin = jnp.maximum(b_ymin, cy)
        ixmax = jnp.minimum(b_xmax, cxx)
        iymax = jnp.minimum(b_ymax, cyy)
        iw = jnp.maximum(ixmax - ixmin, 0.0)
        ih = jnp.maximum(iymax - iymin, 0.0)
        inter = iw * ih
        area1 = (jnp.maximum(b_xmax - b_xmin, 0.0)
                 * jnp.maximum(b_ymax - b_ymin, 0.0))
        union = area1 + carea_ref[:, :] - inter
        iou = inter / jnp.maximum(union, 1e-8)
        csc_ref[:, :] = jnp.where(iou >= NMST, 0.0, csc)

        sel = (lane == i) & valid
        out_ref[0, :, :] = jnp.where(sel, b_xmin, out_ref[0, :, :])
        out_ref[1, :, :] = jnp.where(sel, b_ymin, out_ref[1, :, :])
        out_ref[2, :, :] = jnp.where(sel, b_xmax, out_ref[2, :, :])
        out_ref[3, :, :] = jnp.where(sel, b_ymax, out_ref[3, :, :])
        out_ref[4, :, :] = jnp.where(sel, m, out_ref[4, :, :])
        return count + valid.astype(jnp.int32)

    count0 = jnp.zeros((NCLS, 1), jnp.int32)
    count = lax.fori_loop(0, MAXP, body, count0)

    flags = count < MAXP

    @pl.when(jnp.any(flags))
    def _():
        sfull_ref[:, :] = jnp.where(flags, scores_in_ref[:, :], 0.0)
        flag_l = jnp.broadcast_to(flags, (NCLS, PADP))
        out_ref[0, :, :] = jnp.where(flag_l, 0.0, out_ref[0, :, :])
        out_ref[1, :, :] = jnp.where(flag_l, 0.0, out_ref[1, :, :])
        out_ref[2, :, :] = jnp.where(flag_l, 0.0, out_ref[2, :, :])
        out_ref[3, :, :] = jnp.where(flag_l, 0.0, out_ref[3, :, :])
        out_ref[4, :, :] = jnp.where(flag_l, 0.0, out_ref[4, :, :])

        iota_n = lax.broadcasted_iota(jnp.int32, (NCLS, N), 1)

        def step(i, _):
            scores = sfull_ref[:, :]
            xmin = boxes_ref[0:1, :]
            ymin = boxes_ref[1:2, :]
            xmax = boxes_ref[2:3, :]
            ymax = boxes_ref[3:4, :]

            m = jnp.max(scores, axis=1, keepdims=True)
            idx = jnp.min(jnp.where(scores == m, iota_n, N), axis=1,
                          keepdims=True)
            onehot = (iota_n == idx).astype(jnp.float32)
            b_xmin = jnp.sum(onehot * xmin, axis=1, keepdims=True)
            b_ymin = jnp.sum(onehot * ymin, axis=1, keepdims=True)
            b_xmax = jnp.sum(onehot * xmax, axis=1, keepdims=True)
            b_ymax = jnp.sum(onehot * ymax, axis=1, keepdims=True)
            valid = m > 0.0

            ixmin = jnp.maximum(b_xmin, xmin)
            iymin = jnp.maximum(b_ymin, ymin)
            ixmax = jnp.minimum(b_xmax, xmax)
            iymax = jnp.minimum(b_ymax, ymax)
            iw = jnp.maximum(ixmax - ixmin, 0.0)
            ih = jnp.maximum(iymax - iymin, 0.0)
            inter = iw * ih
            area1 = (jnp.maximum(b_xmax - b_xmin, 0.0)
                     * jnp.maximum(b_ymax - b_ymin, 0.0))
            area2 = (jnp.maximum(xmax - xmin, 0.0)
                     * jnp.maximum(ymax - ymin, 0.0))
            union = area1 + area2 - inter
            iou = inter / jnp.maximum(union, 1e-8)
            sfull_ref[:, :] = jnp.where(iou >= NMST, 0.0, scores)

            sel = (lane == i) & valid
            out_ref[0, :, :] = jnp.where(sel, b_xmin, out_ref[0, :, :])
            out_ref[1, :, :] = jnp.where(sel, b_ymin, out_ref[1, :, :])
            out_ref[2, :, :] = jnp.where(sel, b_xmax, out_ref[2, :, :])
            out_ref[3, :, :] = jnp.where(sel, b_ymax, out_ref[3, :, :])
            out_ref[4, :, :] = jnp.where(sel, m, out_ref[4, :, :])
            return 0

        lax.fori_loop(0, MAXP, step, 0)


def kernel(arm_cls, arm_loc, odm_cls, odm_loc, anchors):
    arm_bg = arm_cls[0, :, 0][None, :]
    arm_loc_t = arm_loc[0].T
    odm_cls_t = odm_cls[0].T[1:NCLS + 1]
    odm_loc_t = odm_loc[0].T
    anchors_t = anchors.T

    scores, boxes8 = pl.pallas_call(
        _decode_body,
        out_shape=(
            jax.ShapeDtypeStruct((NCLS, N), jnp.float32),
            jax.ShapeDtypeStruct((8, N), jnp.float32),
        ),
    )(arm_bg, arm_loc_t, odm_cls_t, odm_loc_t, anchors_t)

    cs, cx, cy, cxx, cyy = _sc_compact(scores, boxes8)

    out = pl.pallas_call(
        _nms_body,
        out_shape=jax.ShapeDtypeStruct((5, NCLS, PADP), jnp.float32),
        scratch_shapes=[
            pltpu.VMEM((NCLS, K), jnp.float32),
            pltpu.VMEM((NCLS, K), jnp.float32),
            pltpu.VMEM((NCLS, N), jnp.float32),
        ],
    )(cs, cx, cy, cxx, cyy, scores, boxes8)

    return jnp.transpose(out[:, :, :MAXP], (1, 2, 0))[None]


# speedup vs baseline: 46.0482x; 1.0507x over previous
import functools

import jax
import jax.numpy as jnp
from jax import lax
from jax.experimental import pallas as pl
from jax.experimental.pallas import tpu as pltpu
from jax.experimental.pallas import tpu_sc as plsc

N = 20000
NCLS = 20
MAXP = 200
PADP = 256
CONF_T = 0.01
ART = 0.99
NMST = 0.45
VAR = (0.1, 0.1, 0.2, 0.2)
K = 1280
TAU = 0.95


def _decode_body(arm_bg_ref, arm_loc_ref, odm_cls_ref, odm_loc_ref,
                 anchors_ref, scores_ref, boxes_ref):
    a_cx = anchors_ref[0:1, :]
    a_cy = anchors_ref[1:2, :]
    a_w = anchors_ref[2:3, :]
    a_h = anchors_ref[3:4, :]

    r_cx = arm_loc_ref[0:1, :] * VAR[0] * a_w + a_cx
    r_cy = arm_loc_ref[1:2, :] * VAR[1] * a_h + a_cy
    r_w = a_w * jnp.exp(arm_loc_ref[2:3, :] * VAR[2])
    r_h = a_h * jnp.exp(arm_loc_ref[3:4, :] * VAR[3])

    d_cx = odm_loc_ref[0:1, :] * VAR[0] * r_w + r_cx
    d_cy = odm_loc_ref[1:2, :] * VAR[1] * r_h + r_cy
    d_w = r_w * jnp.exp(odm_loc_ref[2:3, :] * VAR[2])
    d_h = r_h * jnp.exp(odm_loc_ref[3:4, :] * VAR[3])

    xmin = jnp.clip(d_cx - d_w * 0.5, 0.0, 1.0)
    ymin = jnp.clip(d_cy - d_h * 0.5, 0.0, 1.0)
    xmax = jnp.clip(d_cx + d_w * 0.5, 0.0, 1.0)
    ymax = jnp.clip(d_cy + d_h * 0.5, 0.0, 1.0)

    boxes_ref[0:1, :] = xmin
    boxes_ref[1:2, :] = ymin
    boxes_ref[2:3, :] = xmax
    boxes_ref[3:4, :] = ymax
    z = jnp.zeros_like(xmin)
    boxes_ref[4:5, :] = z
    boxes_ref[5:6, :] = z
    boxes_ref[6:7, :] = z
    boxes_ref[7:8, :] = z

    keep = 1.0 - (arm_bg_ref[0:1, :] >= ART).astype(jnp.float32)
    cls = odm_cls_ref[:, :] * keep
    scores_ref[:, :] = jnp.where(cls > CONF_T, cls, 0.0)


_sc_mesh = plsc.VectorSubcoreMesh(core_axis_name="c", subcore_axis_name="s")


@functools.partial(
    pl.kernel,
    mesh=_sc_mesh,
    compiler_params=pltpu.CompilerParams(needs_layout_passes=False),
    out_type=(
        jax.ShapeDtypeStruct((NCLS, K), jnp.float32),
        jax.ShapeDtypeStruct((NCLS, K), jnp.float32),
        jax.ShapeDtypeStruct((NCLS, K), jnp.float32),
        jax.ShapeDtypeStruct((NCLS, K), jnp.float32),
        jax.ShapeDtypeStruct((NCLS, K), jnp.float32),
    ),
    scratch_types=[
        pltpu.VMEM((N,), jnp.float32),
        pltpu.VMEM((N,), jnp.float32),
        pltpu.VMEM((N,), jnp.float32),
        pltpu.VMEM((N,), jnp.float32),
        pltpu.VMEM((N,), jnp.float32),
        pltpu.VMEM((K,), jnp.float32),
        pltpu.VMEM((K,), jnp.int32),
        pltpu.VMEM((K,), jnp.float32),
    ],
)
def _sc_compact(scores_hbm, boxes_hbm,
                cs_hbm, cx_hbm, cy_hbm, cxx_hbm, cyy_hbm,
                s_v, x0_v, y0_v, x1_v, y1_v, cs_v, ci_v, plane_v):
    wid = lax.axis_index("c") * 16 + lax.axis_index("s")

    @pl.when(wid < NCLS)
    def _():
        pltpu.sync_copy(scores_hbm.at[wid], s_v)
        pltpu.sync_copy(boxes_hbm.at[0], x0_v)
        pltpu.sync_copy(boxes_hbm.at[1], y0_v)
        pltpu.sync_copy(boxes_hbm.at[2], x1_v)
        pltpu.sync_copy(boxes_hbm.at[3], y1_v)

        zf = jnp.zeros((16,), jnp.float32)
        zi = jnp.zeros((16,), jnp.int32)

        def zero_loop(j, _):
            cs_v[pl.ds(j * 16, 16)] = zf
            ci_v[pl.ds(j * 16, 16)] = zi
            return 0

        lax.fori_loop(0, K // 16, zero_loop, 0)

        lanes = lax.iota(jnp.int32, 16)

        def scan(i, off):
            for u in range(5):
                c = i * 5 + u
                v = s_v[pl.ds(c * 16, 16)]
                msk = v > TAU
                cnt = jnp.sum(jnp.where(msk, jnp.ones((16,), jnp.int32),
                                        jnp.zeros((16,), jnp.int32)))
                base = jnp.minimum(off, K - 16)
                plsc.store_compressed(cs_v.at[pl.ds(base, 16)], v, mask=msk)
                plsc.store_compressed(ci_v.at[pl.ds(base, 16)],
                                      lanes + c * 16, mask=msk)
                off = off + cnt
            return off

        off_final = lax.fori_loop(0, N // 80, scan, 0)

        @pl.when(off_final > K - 16)
        def _():
            lax.fori_loop(0, K // 16, zero_loop, 0)

        pltpu.sync_copy(cs_v, cs_hbm.at[wid])

        for coord_v, out_hbm in ((x0_v, cx_hbm), (y0_v, cy_hbm),
                                 (x1_v, cxx_hbm), (y1_v, cyy_hbm)):
            def extract(j, _, coord_v=coord_v):
                idxv = ci_v[pl.ds(j * 16, 16)]
                plane_v[pl.ds(j * 16, 16)] = plsc.load_gather(coord_v, [idxv])
                return 0

            lax.fori_loop(0, K // 16, extract, 0)
            pltpu.sync_copy(plane_v, out_hbm.at[wid])


def _nms_body(cs_ref, cx_ref, cy_ref, cxx_ref, cyy_ref, scores_in_ref,
              boxes_ref, out_ref, csc_ref, carea_ref, sfull_ref):
    csc_ref[:, :] = cs_ref[:, :]
    carea_ref[:, :] = (jnp.maximum(cxx_ref[:, :] - cx_ref[:, :], 0.0)
                       * jnp.maximum(cyy_ref[:, :] - cy_ref[:, :], 0.0))
    out_ref[:, :, :] = jnp.zeros((5, NCLS, PADP), jnp.float32)

    iota_k = lax.broadcasted_iota(jnp.int32, (NCLS, K), 1)
    lane = lax.broadcasted_iota(jnp.int32, (NCLS, PADP), 1)

    def body(i, count):
        csc = csc_ref[:, :]
        cx = cx_ref[:, :]
        cy = cy_ref[:, :]
        cxx = cxx_ref[:, :]
        cyy = cyy_ref[:, :]
        m = jnp.max(csc, axis=1, keepdims=True)
        idx = jnp.min(jnp.where(csc == m, iota_k, K), axis=1,
                      keepdims=True)
        onehot = iota_k == idx
        b_xmin = jnp.sum(jnp.where(onehot, cx, 0.0), axis=1,
                         keepdims=True)
        b_ymin = jnp.sum(jnp.where(onehot, cy, 0.0), axis=1,
                         keepdims=True)
        b_xmax = jnp.sum(jnp.where(onehot, cxx, 0.0), axis=1,
                         keepdims=True)
        b_ymax = jnp.sum(jnp.where(onehot, cyy, 0.0), axis=1,
                         keepdims=True)
        valid = m > 0.0

        ixmin = jnp.maximum(b_xmin, cx)
        iymin = jnp.maximum(b_ymin, cy)
        ixmax = jnp.minimum(b_xmax, cxx)
        iymax = jnp.minimum(b_ymax, cyy)
        iw = jnp.maximum(ixmax - ixmin, 0.0)
        ih = jnp.maximum(iymax - iymin, 0.0)
        inter = iw * ih
        area1 = (jnp.maximum(b_xmax - b_xmin, 0.0)
                 * jnp.maximum(b_ymax - b_ymin, 0.0))
        union = area1 + carea_ref[:, :] - inter
        iou = inter / jnp.maximum(union, 1e-8)
        csc_ref[:, :] = jnp.where(iou >= NMST, 0.0, csc)

        sel = (lane == i) & valid
        out_ref[0, :, :] = jnp.where(sel, b_xmin, out_ref[0, :, :])
        out_ref[1, :, :] = jnp.where(sel, b_ymin, out_ref[1, :, :])
        out_ref[2, :, :] = jnp.where(sel, b_xmax, out_ref[2, :, :])
        out_ref[3, :, :] = jnp.where(sel, b_ymax, out_ref[3, :, :])
        out_ref[4, :, :] = jnp.where(sel, m, out_ref[4, :, :])
        return count + valid.astype(jnp.int32)

    count0 = jnp.zeros((NCLS, 1), jnp.int32)
    count = lax.fori_loop(0, MAXP, body, count0, unroll=2)

    flags = count < MAXP

    @pl.when(jnp.any(flags))
    def _():
        sfull_ref[:, :] = jnp.where(flags, scores_in_ref[:, :], 0.0)
        flag_l = jnp.broadcast_to(flags, (NCLS, PADP))
        out_ref[0, :, :] = jnp.where(flag_l, 0.0, out_ref[0, :, :])
        out_ref[1, :, :] = jnp.where(flag_l, 0.0, out_ref[1, :, :])
        out_ref[2, :, :] = jnp.where(flag_l, 0.0, out_ref[2, :, :])
        out_ref[3, :, :] = jnp.where(flag_l, 0.0, out_ref[3, :, :])
        out_ref[4, :, :] = jnp.where(flag_l, 0.0, out_ref[4, :, :])

        iota_n = lax.broadcasted_iota(jnp.int32, (NCLS, N), 1)

        def step(i, _):
            scores = sfull_ref[:, :]
            xmin = boxes_ref[0:1, :]
            ymin = boxes_ref[1:2, :]
            xmax = boxes_ref[2:3, :]
            ymax = boxes_ref[3:4, :]

            m = jnp.max(scores, axis=1, keepdims=True)
            idx = jnp.min(jnp.where(scores == m, iota_n, N), axis=1,
                          keepdims=True)
            onehot = (iota_n == idx).astype(jnp.float32)
            b_xmin = jnp.sum(onehot * xmin, axis=1, keepdims=True)
            b_ymin = jnp.sum(onehot * ymin, axis=1, keepdims=True)
            b_xmax = jnp.sum(onehot * xmax, axis=1, keepdims=True)
            b_ymax = jnp.sum(onehot * ymax, axis=1, keepdims=True)
            valid = m > 0.0

            ixmin = jnp.maximum(b_xmin, xmin)
            iymin = jnp.maximum(b_ymin, ymin)
            ixmax = jnp.minimum(b_xmax, xmax)
            iymax = jnp.minimum(b_ymax, ymax)
            iw = jnp.maximum(ixmax - ixmin, 0.0)
            ih = jnp.maximum(iymax - iymin, 0.0)
            inter = iw * ih
            area1 = (jnp.maximum(b_xmax - b_xmin, 0.0)
                     * jnp.maximum(b_ymax - b_ymin, 0.0))
            area2 = (jnp.maximum(xmax - xmin, 0.0)
                     * jnp.maximum(ymax - ymin, 0.0))
            union = area1 + area2 - inter
            iou = inter / jnp.maximum(union, 1e-8)
            sfull_ref[:, :] = jnp.where(iou >= NMST, 0.0, scores)

            sel = (lane == i) & valid
            out_ref[0, :, :] = jnp.where(sel, b_xmin, out_ref[0, :, :])
            out_ref[1, :, :] = jnp.where(sel, b_ymin, out_ref[1, :, :])
            out_ref[2, :, :] = jnp.where(sel, b_xmax, out_ref[2, :, :])
            out_ref[3, :, :] = jnp.where(sel, b_ymax, out_ref[3, :, :])
            out_ref[4, :, :] = jnp.where(sel, m, out_ref[4, :, :])
            return 0

        lax.fori_loop(0, MAXP, step, 0)


def kernel(arm_cls, arm_loc, odm_cls, odm_loc, anchors):
    arm_bg = arm_cls[0, :, 0][None, :]
    arm_loc_t = arm_loc[0].T
    odm_cls_t = odm_cls[0].T[1:NCLS + 1]
    odm_loc_t = odm_loc[0].T
    anchors_t = anchors.T

    scores, boxes8 = pl.pallas_call(
        _decode_body,
        out_shape=(
            jax.ShapeDtypeStruct((NCLS, N), jnp.float32),
            jax.ShapeDtypeStruct((8, N), jnp.float32),
        ),
    )(arm_bg, arm_loc_t, odm_cls_t, odm_loc_t, anchors_t)

    cs, cx, cy, cxx, cyy = _sc_compact(scores, boxes8)

    out = pl.pallas_call(
        _nms_body,
        out_shape=jax.ShapeDtypeStruct((5, NCLS, PADP), jnp.float32),
        scratch_shapes=[
            pltpu.VMEM((NCLS, K), jnp.float32),
            pltpu.VMEM((NCLS, K), jnp.float32),
            pltpu.VMEM((NCLS, N), jnp.float32),
        ],
    )(cs, cx, cy, cxx, cyy, scores, boxes8)

    return jnp.transpose(out[:, :, :MAXP], (1, 2, 0))[None]


# speedup vs baseline: 48.8752x; 1.0614x over previous
import functools

import jax
import jax.numpy as jnp
from jax import lax
from jax.experimental import pallas as pl
from jax.experimental.pallas import tpu as pltpu
from jax.experimental.pallas import tpu_sc as plsc

N = 20000
NCLS = 20
MAXP = 200
PADP = 256
CONF_T = 0.01
ART = 0.99
NMST = 0.45
VAR = (0.1, 0.1, 0.2, 0.2)
K = 1024
TAU = 0.96


def _decode_body(arm_bg_ref, arm_loc_ref, odm_cls_ref, odm_loc_ref,
                 anchors_ref, scores_ref, boxes_ref):
    a_cx = anchors_ref[0:1, :]
    a_cy = anchors_ref[1:2, :]
    a_w = anchors_ref[2:3, :]
    a_h = anchors_ref[3:4, :]

    r_cx = arm_loc_ref[0:1, :] * VAR[0] * a_w + a_cx
    r_cy = arm_loc_ref[1:2, :] * VAR[1] * a_h + a_cy
    r_w = a_w * jnp.exp(arm_loc_ref[2:3, :] * VAR[2])
    r_h = a_h * jnp.exp(arm_loc_ref[3:4, :] * VAR[3])

    d_cx = odm_loc_ref[0:1, :] * VAR[0] * r_w + r_cx
    d_cy = odm_loc_ref[1:2, :] * VAR[1] * r_h + r_cy
    d_w = r_w * jnp.exp(odm_loc_ref[2:3, :] * VAR[2])
    d_h = r_h * jnp.exp(odm_loc_ref[3:4, :] * VAR[3])

    xmin = jnp.clip(d_cx - d_w * 0.5, 0.0, 1.0)
    ymin = jnp.clip(d_cy - d_h * 0.5, 0.0, 1.0)
    xmax = jnp.clip(d_cx + d_w * 0.5, 0.0, 1.0)
    ymax = jnp.clip(d_cy + d_h * 0.5, 0.0, 1.0)

    boxes_ref[0:1, :] = xmin
    boxes_ref[1:2, :] = ymin
    boxes_ref[2:3, :] = xmax
    boxes_ref[3:4, :] = ymax
    z = jnp.zeros_like(xmin)
    boxes_ref[4:5, :] = z
    boxes_ref[5:6, :] = z
    boxes_ref[6:7, :] = z
    boxes_ref[7:8, :] = z

    keep = 1.0 - (arm_bg_ref[0:1, :] >= ART).astype(jnp.float32)
    cls = odm_cls_ref[:, :] * keep
    scores_ref[:, :] = jnp.where(cls > CONF_T, cls, 0.0)


_sc_mesh = plsc.VectorSubcoreMesh(core_axis_name="c", subcore_axis_name="s")


@functools.partial(
    pl.kernel,
    mesh=_sc_mesh,
    compiler_params=pltpu.CompilerParams(needs_layout_passes=False),
    out_type=(
        jax.ShapeDtypeStruct((NCLS, K), jnp.float32),
        jax.ShapeDtypeStruct((NCLS, K), jnp.float32),
        jax.ShapeDtypeStruct((NCLS, K), jnp.float32),
        jax.ShapeDtypeStruct((NCLS, K), jnp.float32),
        jax.ShapeDtypeStruct((NCLS, K), jnp.float32),
    ),
    scratch_types=[
        pltpu.VMEM((N,), jnp.float32),
        pltpu.VMEM((N,), jnp.float32),
        pltpu.VMEM((N,), jnp.float32),
        pltpu.VMEM((N,), jnp.float32),
        pltpu.VMEM((N,), jnp.float32),
        pltpu.VMEM((K,), jnp.float32),
        pltpu.VMEM((K,), jnp.int32),
        pltpu.VMEM((K,), jnp.float32),
    ],
)
def _sc_compact(scores_hbm, boxes_hbm,
                cs_hbm, cx_hbm, cy_hbm, cxx_hbm, cyy_hbm,
                s_v, x0_v, y0_v, x1_v, y1_v, cs_v, ci_v, plane_v):
    wid = lax.axis_index("c") * 16 + lax.axis_index("s")

    @pl.when(wid < NCLS)
    def _():
        pltpu.sync_copy(scores_hbm.at[wid], s_v)
        pltpu.sync_copy(boxes_hbm.at[0], x0_v)
        pltpu.sync_copy(boxes_hbm.at[1], y0_v)
        pltpu.sync_copy(boxes_hbm.at[2], x1_v)
        pltpu.sync_copy(boxes_hbm.at[3], y1_v)

        zf = jnp.zeros((16,), jnp.float32)
        zi = jnp.zeros((16,), jnp.int32)

        def zero_loop(j, _):
            cs_v[pl.ds(j * 16, 16)] = zf
            ci_v[pl.ds(j * 16, 16)] = zi
            return 0

        lax.fori_loop(0, K // 16, zero_loop, 0)

        lanes = lax.iota(jnp.int32, 16)

        def scan(i, off):
            for u in range(5):
                c = i * 5 + u
                v = s_v[pl.ds(c * 16, 16)]
                msk = v > TAU
                cnt = jnp.sum(jnp.where(msk, jnp.ones((16,), jnp.int32),
                                        jnp.zeros((16,), jnp.int32)))
                base = jnp.minimum(off, K - 16)
                plsc.store_compressed(cs_v.at[pl.ds(base, 16)], v, mask=msk)
                plsc.store_compressed(ci_v.at[pl.ds(base, 16)],
                                      lanes + c * 16, mask=msk)
                off = off + cnt
            return off

        off_final = lax.fori_loop(0, N // 80, scan, 0)

        @pl.when(off_final > K - 16)
        def _():
            lax.fori_loop(0, K // 16, zero_loop, 0)

        pltpu.sync_copy(cs_v, cs_hbm.at[wid])

        for coord_v, out_hbm in ((x0_v, cx_hbm), (y0_v, cy_hbm),
                                 (x1_v, cxx_hbm), (y1_v, cyy_hbm)):
            def extract(j, _, coord_v=coord_v):
                idxv = ci_v[pl.ds(j * 16, 16)]
                plane_v[pl.ds(j * 16, 16)] = plsc.load_gather(coord_v, [idxv])
                return 0

            lax.fori_loop(0, K // 16, extract, 0)
            pltpu.sync_copy(plane_v, out_hbm.at[wid])


def _nms_body(cs_ref, cx_ref, cy_ref, cxx_ref, cyy_ref, scores_in_ref,
              boxes_ref, out_ref, csc_ref, carea_ref, sfull_ref):
    csc_ref[:, :] = cs_ref[:, :]
    carea_ref[:, :] = (jnp.maximum(cxx_ref[:, :] - cx_ref[:, :], 0.0)
                       * jnp.maximum(cyy_ref[:, :] - cy_ref[:, :], 0.0))
    out_ref[:, :, :] = jnp.zeros((5, NCLS, PADP), jnp.float32)

    iota_k = lax.broadcasted_iota(jnp.int32, (NCLS, K), 1)
    lane = lax.broadcasted_iota(jnp.int32, (NCLS, PADP), 1)

    csc0 = csc_ref[:, :]
    m0 = jnp.max(csc0, axis=1, keepdims=True)
    idx0 = jnp.min(jnp.where(csc0 == m0, iota_k, K), axis=1, keepdims=True)

    def body(i, carry):
        count, m, idx = carry
        cx = cx_ref[:, :]
        cy = cy_ref[:, :]
        cxx = cxx_ref[:, :]
        cyy = cyy_ref[:, :]
        onehot = iota_k == idx
        b_xmin = jnp.sum(jnp.where(onehot, cx, 0.0), axis=1,
                         keepdims=True)
        b_ymin = jnp.sum(jnp.where(onehot, cy, 0.0), axis=1,
                         keepdims=True)
        b_xmax = jnp.sum(jnp.where(onehot, cxx, 0.0), axis=1,
                         keepdims=True)
        b_ymax = jnp.sum(jnp.where(onehot, cyy, 0.0), axis=1,
                         keepdims=True)
        valid = m > 0.0

        csc = csc_ref[:, :]
        ixmin = jnp.maximum(b_xmin, cx)
        iymin = jnp.maximum(b_ymin, cy)
        ixmax = jnp.minimum(b_xmax, cxx)
        iymax = jnp.minimum(b_ymax, cyy)
        iw = jnp.maximum(ixmax - ixmin, 0.0)
        ih = jnp.maximum(iymax - iymin, 0.0)
        inter = iw * ih
        area1 = (jnp.maximum(b_xmax - b_xmin, 0.0)
                 * jnp.maximum(b_ymax - b_ymin, 0.0))
        union = area1 + carea_ref[:, :] - inter
        iou = inter / jnp.maximum(union, 1e-8)
        new_csc = jnp.where(iou >= NMST, 0.0, csc)
        csc_ref[:, :] = new_csc
        m2 = jnp.max(new_csc, axis=1, keepdims=True)
        idx2 = jnp.min(jnp.where(new_csc == m2, iota_k, K), axis=1,
                       keepdims=True)

        sel = (lane == i) & valid
        out_ref[0, :, :] = jnp.where(sel, b_xmin, out_ref[0, :, :])
        out_ref[1, :, :] = jnp.where(sel, b_ymin, out_ref[1, :, :])
        out_ref[2, :, :] = jnp.where(sel, b_xmax, out_ref[2, :, :])
        out_ref[3, :, :] = jnp.where(sel, b_ymax, out_ref[3, :, :])
        out_ref[4, :, :] = jnp.where(sel, m, out_ref[4, :, :])
        return count + valid.astype(jnp.int32), m2, idx2

    count0 = jnp.zeros((NCLS, 1), jnp.int32)
    count, _, _ = lax.fori_loop(0, MAXP, body, (count0, m0, idx0),
                                unroll=2)

    flags = count < MAXP

    @pl.when(jnp.any(flags))
    def _():
        sfull_ref[:, :] = jnp.where(flags, scores_in_ref[:, :], 0.0)
        flag_l = jnp.broadcast_to(flags, (NCLS, PADP))
        out_ref[0, :, :] = jnp.where(flag_l, 0.0, out_ref[0, :, :])
        out_ref[1, :, :] = jnp.where(flag_l, 0.0, out_ref[1, :, :])
        out_ref[2, :, :] = jnp.where(flag_l, 0.0, out_ref[2, :, :])
        out_ref[3, :, :] = jnp.where(flag_l, 0.0, out_ref[3, :, :])
        out_ref[4, :, :] = jnp.where(flag_l, 0.0, out_ref[4, :, :])

        iota_n = lax.broadcasted_iota(jnp.int32, (NCLS, N), 1)

        def step(i, _):
            scores = sfull_ref[:, :]
            xmin = boxes_ref[0:1, :]
            ymin = boxes_ref[1:2, :]
            xmax = boxes_ref[2:3, :]
            ymax = boxes_ref[3:4, :]

            m = jnp.max(scores, axis=1, keepdims=True)
            idx = jnp.min(jnp.where(scores == m, iota_n, N), axis=1,
                          keepdims=True)
            onehot = (iota_n == idx).astype(jnp.float32)
            b_xmin = jnp.sum(onehot * xmin, axis=1, keepdims=True)
            b_ymin = jnp.sum(onehot * ymin, axis=1, keepdims=True)
            b_xmax = jnp.sum(onehot * xmax, axis=1, keepdims=True)
            b_ymax = jnp.sum(onehot * ymax, axis=1, keepdims=True)
            valid = m > 0.0

            ixmin = jnp.maximum(b_xmin, xmin)
            iymin = jnp.maximum(b_ymin, ymin)
            ixmax = jnp.minimum(b_xmax, xmax)
            iymax = jnp.minimum(b_ymax, ymax)
            iw = jnp.maximum(ixmax - ixmin, 0.0)
            ih = jnp.maximum(iymax - iymin, 0.0)
            inter = iw * ih
            area1 = (jnp.maximum(b_xmax - b_xmin, 0.0)
                     * jnp.maximum(b_ymax - b_ymin, 0.0))
            area2 = (jnp.maximum(xmax - xmin, 0.0)
                     * jnp.maximum(ymax - ymin, 0.0))
            union = area1 + area2 - inter
            iou = inter / jnp.maximum(union, 1e-8)
            sfull_ref[:, :] = jnp.where(iou >= NMST, 0.0, scores)

            sel = (lane == i) & valid
            out_ref[0, :, :] = jnp.where(sel, b_xmin, out_ref[0, :, :])
            out_ref[1, :, :] = jnp.where(sel, b_ymin, out_ref[1, :, :])
            out_ref[2, :, :] = jnp.where(sel, b_xmax, out_ref[2, :, :])
            out_ref[3, :, :] = jnp.where(sel, b_ymax, out_ref[3, :, :])
            out_ref[4, :, :] = jnp.where(sel, m, out_ref[4, :, :])
            return 0

        lax.fori_loop(0, MAXP, step, 0)


def kernel(arm_cls, arm_loc, odm_cls, odm_loc, anchors):
    arm_bg = arm_cls[0, :, 0][None, :]
    arm_loc_t = arm_loc[0].T
    odm_cls_t = odm_cls[0].T[1:NCLS + 1]
    odm_loc_t = odm_loc[0].T
    anchors_t = anchors.T

    scores, boxes8 = pl.pallas_call(
        _decode_body,
        out_shape=(
            jax.ShapeDtypeStruct((NCLS, N), jnp.float32),
            jax.ShapeDtypeStruct((8, N), jnp.float32),
        ),
    )(arm_bg, arm_loc_t, odm_cls_t, odm_loc_t, anchors_t)

    cs, cx, cy, cxx, cyy = _sc_compact(scores, boxes8)

    out = pl.pallas_call(
        _nms_body,
        out_shape=jax.ShapeDtypeStruct((5, NCLS, PADP), jnp.float32),
        scratch_shapes=[
            pltpu.VMEM((NCLS, K), jnp.float32),
            pltpu.VMEM((NCLS, K), jnp.float32),
            pltpu.VMEM((NCLS, N), jnp.float32),
        ],
    )(cs, cx, cy, cxx, cyy, scores, boxes8)

    return jnp.transpose(out[:, :, :MAXP], (1, 2, 0))[None]


# speedup vs baseline: 48.9915x; 1.0024x over previous
import functools

import jax
import jax.numpy as jnp
from jax import lax
from jax.experimental import pallas as pl
from jax.experimental.pallas import tpu as pltpu
from jax.experimental.pallas import tpu_sc as plsc

N = 20000
NCLS = 20
MAXP = 200
PADP = 256
CONF_T = 0.01
ART = 0.99
NMST = 0.45
VAR = (0.1, 0.1, 0.2, 0.2)
K = 1024
TAU = 0.96


def _decode_body(arm_bg_ref, arm_loc_ref, odm_cls_ref, odm_loc_ref,
                 anchors_ref, scores_ref, boxes_ref):
    a_cx = anchors_ref[0:1, :]
    a_cy = anchors_ref[1:2, :]
    a_w = anchors_ref[2:3, :]
    a_h = anchors_ref[3:4, :]

    r_cx = arm_loc_ref[0:1, :] * VAR[0] * a_w + a_cx
    r_cy = arm_loc_ref[1:2, :] * VAR[1] * a_h + a_cy
    r_w = a_w * jnp.exp(arm_loc_ref[2:3, :] * VAR[2])
    r_h = a_h * jnp.exp(arm_loc_ref[3:4, :] * VAR[3])

    d_cx = odm_loc_ref[0:1, :] * VAR[0] * r_w + r_cx
    d_cy = odm_loc_ref[1:2, :] * VAR[1] * r_h + r_cy
    d_w = r_w * jnp.exp(odm_loc_ref[2:3, :] * VAR[2])
    d_h = r_h * jnp.exp(odm_loc_ref[3:4, :] * VAR[3])

    xmin = jnp.clip(d_cx - d_w * 0.5, 0.0, 1.0)
    ymin = jnp.clip(d_cy - d_h * 0.5, 0.0, 1.0)
    xmax = jnp.clip(d_cx + d_w * 0.5, 0.0, 1.0)
    ymax = jnp.clip(d_cy + d_h * 0.5, 0.0, 1.0)

    boxes_ref[0:1, :] = xmin
    boxes_ref[1:2, :] = ymin
    boxes_ref[2:3, :] = xmax
    boxes_ref[3:4, :] = ymax
    z = jnp.zeros_like(xmin)
    boxes_ref[4:5, :] = z
    boxes_ref[5:6, :] = z
    boxes_ref[6:7, :] = z
    boxes_ref[7:8, :] = z

    keep = 1.0 - (arm_bg_ref[0:1, :] >= ART).astype(jnp.float32)
    cls = odm_cls_ref[:, :] * keep
    scores_ref[:, :] = jnp.where(cls > CONF_T, cls, 0.0)


_sc_mesh = plsc.VectorSubcoreMesh(core_axis_name="c", subcore_axis_name="s")


@functools.partial(
    pl.kernel,
    mesh=_sc_mesh,
    compiler_params=pltpu.CompilerParams(needs_layout_passes=False),
    out_type=(
        jax.ShapeDtypeStruct((NCLS, K), jnp.float32),
        jax.ShapeDtypeStruct((NCLS, K), jnp.float32),
        jax.ShapeDtypeStruct((NCLS, K), jnp.float32),
        jax.ShapeDtypeStruct((NCLS, K), jnp.float32),
        jax.ShapeDtypeStruct((NCLS, K), jnp.float32),
    ),
    scratch_types=[
        pltpu.VMEM((N,), jnp.float32),
        pltpu.VMEM((N,), jnp.float32),
        pltpu.VMEM((N,), jnp.float32),
        pltpu.VMEM((N,), jnp.float32),
        pltpu.VMEM((N,), jnp.float32),
        pltpu.VMEM((K,), jnp.float32),
        pltpu.VMEM((K,), jnp.int32),
        pltpu.VMEM((K,), jnp.float32),
    ],
)
def _sc_compact(scores_hbm, boxes_hbm,
                cs_hbm, cx_hbm, cy_hbm, cxx_hbm, cyy_hbm,
                s_v, x0_v, y0_v, x1_v, y1_v, cs_v, ci_v, plane_v):
    wid = lax.axis_index("c") * 16 + lax.axis_index("s")

    @pl.when(wid < NCLS)
    def _():
        pltpu.sync_copy(scores_hbm.at[wid], s_v)
        pltpu.sync_copy(boxes_hbm.at[0], x0_v)
        pltpu.sync_copy(boxes_hbm.at[1], y0_v)
        pltpu.sync_copy(boxes_hbm.at[2], x1_v)
        pltpu.sync_copy(boxes_hbm.at[3], y1_v)

        zf = jnp.zeros((16,), jnp.float32)
        zi = jnp.zeros((16,), jnp.int32)

        def zero_loop(j, _):
            cs_v[pl.ds(j * 16, 16)] = zf
            ci_v[pl.ds(j * 16, 16)] = zi
            return 0

        lax.fori_loop(0, K // 16, zero_loop, 0)

        lanes = lax.iota(jnp.int32, 16)

        def scan(i, off):
            for u in range(5):
                c = i * 5 + u
                v = s_v[pl.ds(c * 16, 16)]
                msk = v > TAU
                cnt = jnp.sum(jnp.where(msk, jnp.ones((16,), jnp.int32),
                                        jnp.zeros((16,), jnp.int32)))
                base = jnp.minimum(off, K - 16)
                plsc.store_compressed(cs_v.at[pl.ds(base, 16)], v, mask=msk)
                plsc.store_compressed(ci_v.at[pl.ds(base, 16)],
                                      lanes + c * 16, mask=msk)
                off = off + cnt
            return off

        off_final = lax.fori_loop(0, N // 80, scan, 0)

        @pl.when(off_final > K - 16)
        def _():
            lax.fori_loop(0, K // 16, zero_loop, 0)

        pltpu.sync_copy(cs_v, cs_hbm.at[wid])

        for coord_v, out_hbm in ((x0_v, cx_hbm), (y0_v, cy_hbm),
                                 (x1_v, cxx_hbm), (y1_v, cyy_hbm)):
            def extract(j, _, coord_v=coord_v):
                idxv = ci_v[pl.ds(j * 16, 16)]
                plane_v[pl.ds(j * 16, 16)] = plsc.load_gather(coord_v, [idxv])
                return 0

            lax.fori_loop(0, K // 16, extract, 0)
            pltpu.sync_copy(plane_v, out_hbm.at[wid])


def _nms_body(cs_ref, cx_ref, cy_ref, cxx_ref, cyy_ref, scores_in_ref,
              boxes_ref, out_ref, csc_ref, carea_ref, sfull_ref):
    csc_ref[:, :] = cs_ref[:, :]
    carea_ref[:, :] = (jnp.maximum(cxx_ref[:, :] - cx_ref[:, :], 0.0)
                       * jnp.maximum(cyy_ref[:, :] - cy_ref[:, :], 0.0))
    out_ref[:, :, :] = jnp.zeros((5, NCLS, PADP), jnp.float32)

    iota_k = lax.broadcasted_iota(jnp.int32, (NCLS, K), 1)
    lane = lax.broadcasted_iota(jnp.int32, (NCLS, PADP), 1)

    csc0 = csc_ref[:, :]
    m0 = jnp.max(csc0, axis=1, keepdims=True)
    idx0 = jnp.min(jnp.where(csc0 == m0, iota_k, K), axis=1, keepdims=True)

    def body(i, carry):
        count, m, idx = carry
        cx = cx_ref[:, :]
        cy = cy_ref[:, :]
        cxx = cxx_ref[:, :]
        cyy = cyy_ref[:, :]
        onehot = iota_k == idx
        b_xmin = jnp.sum(jnp.where(onehot, cx, 0.0), axis=1,
                         keepdims=True)
        b_ymin = jnp.sum(jnp.where(onehot, cy, 0.0), axis=1,
                         keepdims=True)
        b_xmax = jnp.sum(jnp.where(onehot, cxx, 0.0), axis=1,
                         keepdims=True)
        b_ymax = jnp.sum(jnp.where(onehot, cyy, 0.0), axis=1,
                         keepdims=True)
        valid = m > 0.0

        csc = csc_ref[:, :]
        ixmin = jnp.maximum(b_xmin, cx)
        iymin = jnp.maximum(b_ymin, cy)
        ixmax = jnp.minimum(b_xmax, cxx)
        iymax = jnp.minimum(b_ymax, cyy)
        iw = jnp.maximum(ixmax - ixmin, 0.0)
        ih = jnp.maximum(iymax - iymin, 0.0)
        inter = iw * ih
        area1 = (jnp.maximum(b_xmax - b_xmin, 0.0)
                 * jnp.maximum(b_ymax - b_ymin, 0.0))
        union = area1 + carea_ref[:, :] - inter
        iou = inter / jnp.maximum(union, 1e-8)
        new_csc = jnp.where(iou >= NMST, 0.0, csc)
        csc_ref[:, :] = new_csc
        m2 = jnp.max(new_csc, axis=1, keepdims=True)
        idx2 = jnp.min(jnp.where(new_csc == m2, iota_k, K), axis=1,
                       keepdims=True)

        sel = (lane == i) & valid
        out_ref[0, :, :] = jnp.where(sel, b_xmin, out_ref[0, :, :])
        out_ref[1, :, :] = jnp.where(sel, b_ymin, out_ref[1, :, :])
        out_ref[2, :, :] = jnp.where(sel, b_xmax, out_ref[2, :, :])
        out_ref[3, :, :] = jnp.where(sel, b_ymax, out_ref[3, :, :])
        out_ref[4, :, :] = jnp.where(sel, m, out_ref[4, :, :])
        return count + valid.astype(jnp.int32), m2, idx2

    count0 = jnp.zeros((NCLS, 1), jnp.int32)
    count, _, _ = lax.fori_loop(0, MAXP, body, (count0, m0, idx0),
                                unroll=4)

    flags = count < MAXP

    @pl.when(jnp.any(flags))
    def _():
        sfull_ref[:, :] = jnp.where(flags, scores_in_ref[:, :], 0.0)
        flag_l = jnp.broadcast_to(flags, (NCLS, PADP))
        out_ref[0, :, :] = jnp.where(flag_l, 0.0, out_ref[0, :, :])
        out_ref[1, :, :] = jnp.where(flag_l, 0.0, out_ref[1, :, :])
        out_ref[2, :, :] = jnp.where(flag_l, 0.0, out_ref[2, :, :])
        out_ref[3, :, :] = jnp.where(flag_l, 0.0, out_ref[3, :, :])
        out_ref[4, :, :] = jnp.where(flag_l, 0.0, out_ref[4, :, :])

        iota_n = lax.broadcasted_iota(jnp.int32, (NCLS, N), 1)

        def step(i, _):
            scores = sfull_ref[:, :]
            xmin = boxes_ref[0:1, :]
            ymin = boxes_ref[1:2, :]
            xmax = boxes_ref[2:3, :]
            ymax = boxes_ref[3:4, :]

            m = jnp.max(scores, axis=1, keepdims=True)
            idx = jnp.min(jnp.where(scores == m, iota_n, N), axis=1,
                          keepdims=True)
            onehot = (iota_n == idx).astype(jnp.float32)
            b_xmin = jnp.sum(onehot * xmin, axis=1, keepdims=True)
            b_ymin = jnp.sum(onehot * ymin, axis=1, keepdims=True)
            b_xmax = jnp.sum(onehot * xmax, axis=1, keepdims=True)
            b_ymax = jnp.sum(onehot * ymax, axis=1, keepdims=True)
            valid = m > 0.0

            ixmin = jnp.maximum(b_xmin, xmin)
            iymin = jnp.maximum(b_ymin, ymin)
            ixmax = jnp.minimum(b_xmax, xmax)
            iymax = jnp.minimum(b_ymax, ymax)
            iw = jnp.maximum(ixmax - ixmin, 0.0)
            ih = jnp.maximum(iymax - iymin, 0.0)
            inter = iw * ih
            area1 = (jnp.maximum(b_xmax - b_xmin, 0.0)
                     * jnp.maximum(b_ymax - b_ymin, 0.0))
            area2 = (jnp.maximum(xmax - xmin, 0.0)
                     * jnp.maximum(ymax - ymin, 0.0))
            union = area1 + area2 - inter
            iou = inter / jnp.maximum(union, 1e-8)
            sfull_ref[:, :] = jnp.where(iou >= NMST, 0.0, scores)

            sel = (lane == i) & valid
            out_ref[0, :, :] = jnp.where(sel, b_xmin, out_ref[0, :, :])
            out_ref[1, :, :] = jnp.where(sel, b_ymin, out_ref[1, :, :])
            out_ref[2, :, :] = jnp.where(sel, b_xmax, out_ref[2, :, :])
            out_ref[3, :, :] = jnp.where(sel, b_ymax, out_ref[3, :, :])
            out_ref[4, :, :] = jnp.where(sel, m, out_ref[4, :, :])
            return 0

        lax.fori_loop(0, MAXP, step, 0)


def kernel(arm_cls, arm_loc, odm_cls, odm_loc, anchors):
    arm_bg = arm_cls[0, :, 0][None, :]
    arm_loc_t = arm_loc[0].T
    odm_cls_t = odm_cls[0].T[1:NCLS + 1]
    odm_loc_t = odm_loc[0].T
    anchors_t = anchors.T

    scores, boxes8 = pl.pallas_call(
        _decode_body,
        out_shape=(
            jax.ShapeDtypeStruct((NCLS, N), jnp.float32),
            jax.ShapeDtypeStruct((8, N), jnp.float32),
        ),
    )(arm_bg, arm_loc_t, odm_cls_t, odm_loc_t, anchors_t)

    cs, cx, cy, cxx, cyy = _sc_compact(scores, boxes8)

    out = pl.pallas_call(
        _nms_body,
        out_shape=jax.ShapeDtypeStruct((5, NCLS, PADP), jnp.float32),
        scratch_shapes=[
            pltpu.VMEM((NCLS, K), jnp.float32),
            pltpu.VMEM((NCLS, K), jnp.float32),
            pltpu.VMEM((NCLS, N), jnp.float32),
        ],
    )(cs, cx, cy, cxx, cyy, scores, boxes8)

    return jnp.transpose(out[:, :, :MAXP], (1, 2, 0))[None]


# speedup vs baseline: 53.9160x; 1.1005x over previous
import functools

import jax
import jax.numpy as jnp
from jax import lax
from jax.experimental import pallas as pl
from jax.experimental.pallas import tpu as pltpu
from jax.experimental.pallas import tpu_sc as plsc

N = 20000
NCLS = 20
MAXP = 200
PADP = 256
CONF_T = 0.01
ART = 0.99
NMST = 0.45
VAR = (0.1, 0.1, 0.2, 0.2)
K = 1024
TAU = 0.96


def _decode_body(arm_bg_ref, arm_loc_ref, odm_cls_ref, odm_loc_ref,
                 anchors_ref, scores_ref, boxes_ref):
    a_cx = anchors_ref[0:1, :]
    a_cy = anchors_ref[1:2, :]
    a_w = anchors_ref[2:3, :]
    a_h = anchors_ref[3:4, :]

    r_cx = arm_loc_ref[0:1, :] * VAR[0] * a_w + a_cx
    r_cy = arm_loc_ref[1:2, :] * VAR[1] * a_h + a_cy
    r_w = a_w * jnp.exp(arm_loc_ref[2:3, :] * VAR[2])
    r_h = a_h * jnp.exp(arm_loc_ref[3:4, :] * VAR[3])

    d_cx = odm_loc_ref[0:1, :] * VAR[0] * r_w + r_cx
    d_cy = odm_loc_ref[1:2, :] * VAR[1] * r_h + r_cy
    d_w = r_w * jnp.exp(odm_loc_ref[2:3, :] * VAR[2])
    d_h = r_h * jnp.exp(odm_loc_ref[3:4, :] * VAR[3])

    xmin = jnp.clip(d_cx - d_w * 0.5, 0.0, 1.0)
    ymin = jnp.clip(d_cy - d_h * 0.5, 0.0, 1.0)
    xmax = jnp.clip(d_cx + d_w * 0.5, 0.0, 1.0)
    ymax = jnp.clip(d_cy + d_h * 0.5, 0.0, 1.0)

    boxes_ref[0:1, :] = xmin
    boxes_ref[1:2, :] = ymin
    boxes_ref[2:3, :] = xmax
    boxes_ref[3:4, :] = ymax
    z = jnp.zeros_like(xmin)
    boxes_ref[4:5, :] = z
    boxes_ref[5:6, :] = z
    boxes_ref[6:7, :] = z
    boxes_ref[7:8, :] = z

    keep = 1.0 - (arm_bg_ref[0:1, :] >= ART).astype(jnp.float32)
    cls = odm_cls_ref[:, :] * keep
    scores_ref[:, :] = jnp.where(cls > CONF_T, cls, 0.0)


_sc_mesh = plsc.VectorSubcoreMesh(core_axis_name="c", subcore_axis_name="s")


@functools.partial(
    pl.kernel,
    mesh=_sc_mesh,
    compiler_params=pltpu.CompilerParams(needs_layout_passes=False),
    out_type=(
        jax.ShapeDtypeStruct((NCLS, K), jnp.float32),
        jax.ShapeDtypeStruct((NCLS, K), jnp.float32),
        jax.ShapeDtypeStruct((NCLS, K), jnp.float32),
        jax.ShapeDtypeStruct((NCLS, K), jnp.float32),
        jax.ShapeDtypeStruct((NCLS, K), jnp.float32),
    ),
    scratch_types=[
        pltpu.VMEM((N,), jnp.float32),
        pltpu.VMEM((N,), jnp.float32),
        pltpu.VMEM((N,), jnp.float32),
        pltpu.VMEM((N,), jnp.float32),
        pltpu.VMEM((N,), jnp.float32),
        pltpu.VMEM((K,), jnp.float32),
        pltpu.VMEM((K,), jnp.int32),
        pltpu.VMEM((K,), jnp.float32),
    ],
)
def _sc_compact(scores_hbm, boxes_hbm,
                cs_hbm, cx_hbm, cy_hbm, cxx_hbm, cyy_hbm,
                s_v, x0_v, y0_v, x1_v, y1_v, cs_v, ci_v, plane_v):
    wid = lax.axis_index("c") * 16 + lax.axis_index("s")

    @pl.when(wid < NCLS)
    def _():
        pltpu.sync_copy(scores_hbm.at[wid], s_v)
        pltpu.sync_copy(boxes_hbm.at[0], x0_v)
        pltpu.sync_copy(boxes_hbm.at[1], y0_v)
        pltpu.sync_copy(boxes_hbm.at[2], x1_v)
        pltpu.sync_copy(boxes_hbm.at[3], y1_v)

        zf = jnp.zeros((16,), jnp.float32)
        zi = jnp.zeros((16,), jnp.int32)

        def zero_loop(j, _):
            cs_v[pl.ds(j * 16, 16)] = zf
            ci_v[pl.ds(j * 16, 16)] = zi
            return 0

        lax.fori_loop(0, K // 16, zero_loop, 0)

        lanes = lax.iota(jnp.int32, 16)

        def scan(i, off):
            for u in range(5):
                c = i * 5 + u
                v = s_v[pl.ds(c * 16, 16)]
                msk = v > TAU
                cnt = jnp.sum(jnp.where(msk, jnp.ones((16,), jnp.int32),
                                        jnp.zeros((16,), jnp.int32)))
                base = jnp.minimum(off, K - 16)
                plsc.store_compressed(cs_v.at[pl.ds(base, 16)], v, mask=msk)
                plsc.store_compressed(ci_v.at[pl.ds(base, 16)],
                                      lanes + c * 16, mask=msk)
                off = off + cnt
            return off

        off_final = lax.fori_loop(0, N // 80, scan, 0)

        @pl.when(off_final > K - 16)
        def _():
            lax.fori_loop(0, K // 16, zero_loop, 0)

        pltpu.sync_copy(cs_v, cs_hbm.at[wid])

        for coord_v, out_hbm in ((x0_v, cx_hbm), (y0_v, cy_hbm),
                                 (x1_v, cxx_hbm), (y1_v, cyy_hbm)):
            def extract(j, _, coord_v=coord_v):
                idxv = ci_v[pl.ds(j * 16, 16)]
                plane_v[pl.ds(j * 16, 16)] = plsc.load_gather(coord_v, [idxv])
                return 0

            lax.fori_loop(0, K // 16, extract, 0)
            pltpu.sync_copy(plane_v, out_hbm.at[wid])


def _nms_body(cs_ref, cx_ref, cy_ref, cxx_ref, cyy_ref, scores_in_ref,
              boxes_ref, out_ref, csc_ref, carea_ref, sfull_ref):
    csc_ref[:, :] = cs_ref[:, :]
    carea_ref[:, :] = (jnp.maximum(cxx_ref[:, :] - cx_ref[:, :], 0.0)
                       * jnp.maximum(cyy_ref[:, :] - cy_ref[:, :], 0.0))
    out_ref[:, :, :] = jnp.zeros((5, NCLS, PADP), jnp.float32)

    iota_k = lax.broadcasted_iota(jnp.int32, (NCLS, K), 1)
    lane = lax.broadcasted_iota(jnp.int32, (NCLS, PADP), 1)

    csc0 = csc_ref[:, :]
    m0 = jnp.max(csc0, axis=1, keepdims=True)
    idx0 = jnp.argmax(csc0, axis=1)[:, None].astype(jnp.int32)

    def body(i, carry):
        count, m, idx = carry
        cx = cx_ref[:, :]
        cy = cy_ref[:, :]
        cxx = cxx_ref[:, :]
        cyy = cyy_ref[:, :]
        onehot = iota_k == idx
        b_xmin = jnp.sum(jnp.where(onehot, cx, 0.0), axis=1,
                         keepdims=True)
        b_ymin = jnp.sum(jnp.where(onehot, cy, 0.0), axis=1,
                         keepdims=True)
        b_xmax = jnp.sum(jnp.where(onehot, cxx, 0.0), axis=1,
                         keepdims=True)
        b_ymax = jnp.sum(jnp.where(onehot, cyy, 0.0), axis=1,
                         keepdims=True)
        valid = m > 0.0

        csc = csc_ref[:, :]
        ixmin = jnp.maximum(b_xmin, cx)
        iymin = jnp.maximum(b_ymin, cy)
        ixmax = jnp.minimum(b_xmax, cxx)
        iymax = jnp.minimum(b_ymax, cyy)
        iw = jnp.maximum(ixmax - ixmin, 0.0)
        ih = jnp.maximum(iymax - iymin, 0.0)
        inter = iw * ih
        area1 = (jnp.maximum(b_xmax - b_xmin, 0.0)
                 * jnp.maximum(b_ymax - b_ymin, 0.0))
        union = area1 + carea_ref[:, :] - inter
        iou = inter / jnp.maximum(union, 1e-8)
        new_csc = jnp.where(iou >= NMST, 0.0, csc)
        csc_ref[:, :] = new_csc
        m2 = jnp.max(new_csc, axis=1, keepdims=True)
        idx2 = jnp.argmax(new_csc, axis=1)[:, None].astype(jnp.int32)

        sel = (lane == i) & valid
        out_ref[0, :, :] = jnp.where(sel, b_xmin, out_ref[0, :, :])
        out_ref[1, :, :] = jnp.where(sel, b_ymin, out_ref[1, :, :])
        out_ref[2, :, :] = jnp.where(sel, b_xmax, out_ref[2, :, :])
        out_ref[3, :, :] = jnp.where(sel, b_ymax, out_ref[3, :, :])
        out_ref[4, :, :] = jnp.where(sel, m, out_ref[4, :, :])
        return count + valid.astype(jnp.int32), m2, idx2

    count0 = jnp.zeros((NCLS, 1), jnp.int32)
    count, _, _ = lax.fori_loop(0, MAXP, body, (count0, m0, idx0),
                                unroll=4)

    flags = count < MAXP

    @pl.when(jnp.any(flags))
    def _():
        sfull_ref[:, :] = jnp.where(flags, scores_in_ref[:, :], 0.0)
        flag_l = jnp.broadcast_to(flags, (NCLS, PADP))
        out_ref[0, :, :] = jnp.where(flag_l, 0.0, out_ref[0, :, :])
        out_ref[1, :, :] = jnp.where(flag_l, 0.0, out_ref[1, :, :])
        out_ref[2, :, :] = jnp.where(flag_l, 0.0, out_ref[2, :, :])
        out_ref[3, :, :] = jnp.where(flag_l, 0.0, out_ref[3, :, :])
        out_ref[4, :, :] = jnp.where(flag_l, 0.0, out_ref[4, :, :])

        iota_n = lax.broadcasted_iota(jnp.int32, (NCLS, N), 1)

        def step(i, _):
            scores = sfull_ref[:, :]
            xmin = boxes_ref[0:1, :]
            ymin = boxes_ref[1:2, :]
            xmax = boxes_ref[2:3, :]
            ymax = boxes_ref[3:4, :]

            m = jnp.max(scores, axis=1, keepdims=True)
            idx = jnp.min(jnp.where(scores == m, iota_n, N), axis=1,
                          keepdims=True)
            onehot = (iota_n == idx).astype(jnp.float32)
            b_xmin = jnp.sum(onehot * xmin, axis=1, keepdims=True)
            b_ymin = jnp.sum(onehot * ymin, axis=1, keepdims=True)
            b_xmax = jnp.sum(onehot * xmax, axis=1, keepdims=True)
            b_ymax = jnp.sum(onehot * ymax, axis=1, keepdims=True)
            valid = m > 0.0

            ixmin = jnp.maximum(b_xmin, xmin)
            iymin = jnp.maximum(b_ymin, ymin)
            ixmax = jnp.minimum(b_xmax, xmax)
            iymax = jnp.minimum(b_ymax, ymax)
            iw = jnp.maximum(ixmax - ixmin, 0.0)
            ih = jnp.maximum(iymax - iymin, 0.0)
            inter = iw * ih
            area1 = (jnp.maximum(b_xmax - b_xmin, 0.0)
                     * jnp.maximum(b_ymax - b_ymin, 0.0))
            area2 = (jnp.maximum(xmax - xmin, 0.0)
                     * jnp.maximum(ymax - ymin, 0.0))
            union = area1 + area2 - inter
            iou = inter / jnp.maximum(union, 1e-8)
            sfull_ref[:, :] = jnp.where(iou >= NMST, 0.0, scores)

            sel = (lane == i) & valid
            out_ref[0, :, :] = jnp.where(sel, b_xmin, out_ref[0, :, :])
            out_ref[1, :, :] = jnp.where(sel, b_ymin, out_ref[1, :, :])
            out_ref[2, :, :] = jnp.where(sel, b_xmax, out_ref[2, :, :])
            out_ref[3, :, :] = jnp.where(sel, b_ymax, out_ref[3, :, :])
            out_ref[4, :, :] = jnp.where(sel, m, out_ref[4, :, :])
            return 0

        lax.fori_loop(0, MAXP, step, 0)


def kernel(arm_cls, arm_loc, odm_cls, odm_loc, anchors):
    arm_bg = arm_cls[0, :, 0][None, :]
    arm_loc_t = arm_loc[0].T
    odm_cls_t = odm_cls[0].T[1:NCLS + 1]
    odm_loc_t = odm_loc[0].T
    anchors_t = anchors.T

    scores, boxes8 = pl.pallas_call(
        _decode_body,
        out_shape=(
            jax.ShapeDtypeStruct((NCLS, N), jnp.float32),
            jax.ShapeDtypeStruct((8, N), jnp.float32),
        ),
    )(arm_bg, arm_loc_t, odm_cls_t, odm_loc_t, anchors_t)

    cs, cx, cy, cxx, cyy = _sc_compact(scores, boxes8)

    out = pl.pallas_call(
        _nms_body,
        out_shape=jax.ShapeDtypeStruct((5, NCLS, PADP), jnp.float32),
        scratch_shapes=[
            pltpu.VMEM((NCLS, K), jnp.float32),
            pltpu.VMEM((NCLS, K), jnp.float32),
            pltpu.VMEM((NCLS, N), jnp.float32),
        ],
    )(cs, cx, cy, cxx, cyy, scores, boxes8)

    return jnp.transpose(out[:, :, :MAXP], (1, 2, 0))[None]


# speedup vs baseline: 55.4755x; 1.0289x over previous
import functools

import jax
import jax.numpy as jnp
from jax import lax
from jax.experimental import pallas as pl
from jax.experimental.pallas import tpu as pltpu
from jax.experimental.pallas import tpu_sc as plsc

N = 20000
NCLS = 20
MAXP = 200
PADP = 256
CONF_T = 0.01
ART = 0.99
NMST = 0.45
VAR = (0.1, 0.1, 0.2, 0.2)
K = 1024
TAU = 0.96
KEY_BASE = 0x3F75C28F
KEY_NEG = -(2 ** 30)


def _decode_body(arm_bg_ref, arm_loc_ref, odm_cls_ref, odm_loc_ref,
                 anchors_ref, scores_ref, boxes_ref):
    a_cx = anchors_ref[0:1, :]
    a_cy = anchors_ref[1:2, :]
    a_w = anchors_ref[2:3, :]
    a_h = anchors_ref[3:4, :]

    r_cx = arm_loc_ref[0:1, :] * VAR[0] * a_w + a_cx
    r_cy = arm_loc_ref[1:2, :] * VAR[1] * a_h + a_cy
    r_w = a_w * jnp.exp(arm_loc_ref[2:3, :] * VAR[2])
    r_h = a_h * jnp.exp(arm_loc_ref[3:4, :] * VAR[3])

    d_cx = odm_loc_ref[0:1, :] * VAR[0] * r_w + r_cx
    d_cy = odm_loc_ref[1:2, :] * VAR[1] * r_h + r_cy
    d_w = r_w * jnp.exp(odm_loc_ref[2:3, :] * VAR[2])
    d_h = r_h * jnp.exp(odm_loc_ref[3:4, :] * VAR[3])

    xmin = jnp.clip(d_cx - d_w * 0.5, 0.0, 1.0)
    ymin = jnp.clip(d_cy - d_h * 0.5, 0.0, 1.0)
    xmax = jnp.clip(d_cx + d_w * 0.5, 0.0, 1.0)
    ymax = jnp.clip(d_cy + d_h * 0.5, 0.0, 1.0)

    boxes_ref[0:1, :] = xmin
    boxes_ref[1:2, :] = ymin
    boxes_ref[2:3, :] = xmax
    boxes_ref[3:4, :] = ymax
    z = jnp.zeros_like(xmin)
    boxes_ref[4:5, :] = z
    boxes_ref[5:6, :] = z
    boxes_ref[6:7, :] = z
    boxes_ref[7:8, :] = z

    keep = 1.0 - (arm_bg_ref[0:1, :] >= ART).astype(jnp.float32)
    cls = odm_cls_ref[:, :] * keep
    scores_ref[:, :] = jnp.where(cls > CONF_T, cls, 0.0)


_sc_mesh = plsc.VectorSubcoreMesh(core_axis_name="c", subcore_axis_name="s")


@functools.partial(
    pl.kernel,
    mesh=_sc_mesh,
    compiler_params=pltpu.CompilerParams(needs_layout_passes=False),
    out_type=(
        jax.ShapeDtypeStruct((NCLS, K), jnp.float32),
        jax.ShapeDtypeStruct((NCLS, K), jnp.float32),
        jax.ShapeDtypeStruct((NCLS, K), jnp.float32),
        jax.ShapeDtypeStruct((NCLS, K), jnp.float32),
        jax.ShapeDtypeStruct((NCLS, K), jnp.float32),
    ),
    scratch_types=[
        pltpu.VMEM((N,), jnp.float32),
        pltpu.VMEM((N,), jnp.float32),
        pltpu.VMEM((N,), jnp.float32),
        pltpu.VMEM((N,), jnp.float32),
        pltpu.VMEM((N,), jnp.float32),
        pltpu.VMEM((K,), jnp.float32),
        pltpu.VMEM((K,), jnp.int32),
        pltpu.VMEM((K,), jnp.float32),
    ],
)
def _sc_compact(scores_hbm, boxes_hbm,
                cs_hbm, cx_hbm, cy_hbm, cxx_hbm, cyy_hbm,
                s_v, x0_v, y0_v, x1_v, y1_v, cs_v, ci_v, plane_v):
    wid = lax.axis_index("c") * 16 + lax.axis_index("s")

    @pl.when(wid < NCLS)
    def _():
        pltpu.sync_copy(scores_hbm.at[wid], s_v)
        pltpu.sync_copy(boxes_hbm.at[0], x0_v)
        pltpu.sync_copy(boxes_hbm.at[1], y0_v)
        pltpu.sync_copy(boxes_hbm.at[2], x1_v)
        pltpu.sync_copy(boxes_hbm.at[3], y1_v)

        zf = jnp.zeros((16,), jnp.float32)
        zi = jnp.zeros((16,), jnp.int32)

        def zero_loop(j, _):
            cs_v[pl.ds(j * 16, 16)] = zf
            ci_v[pl.ds(j * 16, 16)] = zi
            return 0

        lax.fori_loop(0, K // 16, zero_loop, 0)

        lanes = lax.iota(jnp.int32, 16)

        def scan(i, off):
            for u in range(5):
                c = i * 5 + u
                v = s_v[pl.ds(c * 16, 16)]
                msk = v > TAU
                cnt = jnp.sum(jnp.where(msk, jnp.ones((16,), jnp.int32),
                                        jnp.zeros((16,), jnp.int32)))
                base = jnp.minimum(off, K - 16)
                plsc.store_compressed(cs_v.at[pl.ds(base, 16)], v, mask=msk)
                plsc.store_compressed(ci_v.at[pl.ds(base, 16)],
                                      lanes + c * 16, mask=msk)
                off = off + cnt
            return off

        off_final = lax.fori_loop(0, N // 80, scan, 0)

        @pl.when(off_final > K - 16)
        def _():
            lax.fori_loop(0, K // 16, zero_loop, 0)

        pltpu.sync_copy(cs_v, cs_hbm.at[wid])

        for coord_v, out_hbm in ((x0_v, cx_hbm), (y0_v, cy_hbm),
                                 (x1_v, cxx_hbm), (y1_v, cyy_hbm)):
            def extract(j, _, coord_v=coord_v):
                idxv = ci_v[pl.ds(j * 16, 16)]
                plane_v[pl.ds(j * 16, 16)] = plsc.load_gather(coord_v, [idxv])
                return 0

            lax.fori_loop(0, K // 16, extract, 0)
            pltpu.sync_copy(plane_v, out_hbm.at[wid])


def _nms_body(cs_ref, cx_ref, cy_ref, cxx_ref, cyy_ref, scores_in_ref,
              boxes_ref, out_ref, key_ref, carea_ref, sfull_ref):
    carea_ref[:, :] = (jnp.maximum(cxx_ref[:, :] - cx_ref[:, :], 0.0)
                       * jnp.maximum(cyy_ref[:, :] - cy_ref[:, :], 0.0))
    out_ref[:, :, :] = jnp.zeros((5, NCLS, PADP), jnp.float32)

    iota_k = lax.broadcasted_iota(jnp.int32, (NCLS, K), 1)
    lane = lax.broadcasted_iota(jnp.int32, (NCLS, PADP), 1)

    cs0 = cs_ref[:, :]
    csbits = jax.lax.bitcast_convert_type(cs0, jnp.int32)
    revlane = (K - 1) - iota_k
    key0 = jnp.where(cs0 > 0.0,
                     jax.lax.shift_left(csbits - KEY_BASE, 10) + revlane,
                     KEY_NEG)
    key_ref[:, :] = key0
    maxkey0 = jnp.max(key0, axis=1, keepdims=True)

    def body(i, carry):
        count, maxkey = carry
        cx = cx_ref[:, :]
        cy = cy_ref[:, :]
        cxx = cxx_ref[:, :]
        cyy = cyy_ref[:, :]
        valid = maxkey > 0
        idx = (K - 1) - jnp.bitwise_and(maxkey, 1023)
        m = jax.lax.bitcast_convert_type(
            jax.lax.shift_right_logical(maxkey, 10) + KEY_BASE,
            jnp.float32)
        onehot = iota_k == idx
        b_xmin = jnp.sum(jnp.where(onehot, cx, 0.0), axis=1,
                         keepdims=True)
        b_ymin = jnp.sum(jnp.where(onehot, cy, 0.0), axis=1,
                         keepdims=True)
        b_xmax = jnp.sum(jnp.where(onehot, cxx, 0.0), axis=1,
                         keepdims=True)
        b_ymax = jnp.sum(jnp.where(onehot, cyy, 0.0), axis=1,
                         keepdims=True)

        key = key_ref[:, :]
        ixmin = jnp.maximum(b_xmin, cx)
        iymin = jnp.maximum(b_ymin, cy)
        ixmax = jnp.minimum(b_xmax, cxx)
        iymax = jnp.minimum(b_ymax, cyy)
        iw = jnp.maximum(ixmax - ixmin, 0.0)
        ih = jnp.maximum(iymax - iymin, 0.0)
        inter = iw * ih
        area1 = (jnp.maximum(b_xmax - b_xmin, 0.0)
                 * jnp.maximum(b_ymax - b_ymin, 0.0))
        union = area1 + carea_ref[:, :] - inter
        iou = inter / jnp.maximum(union, 1e-8)
        new_key = jnp.where(iou >= NMST, KEY_NEG, key)
        key_ref[:, :] = new_key
        maxkey2 = jnp.max(new_key, axis=1, keepdims=True)

        sel = (lane == i) & valid
        out_ref[0, :, :] = jnp.where(sel, b_xmin, out_ref[0, :, :])
        out_ref[1, :, :] = jnp.where(sel, b_ymin, out_ref[1, :, :])
        out_ref[2, :, :] = jnp.where(sel, b_xmax, out_ref[2, :, :])
        out_ref[3, :, :] = jnp.where(sel, b_ymax, out_ref[3, :, :])
        out_ref[4, :, :] = jnp.where(sel, m, out_ref[4, :, :])
        return count + valid.astype(jnp.int32), maxkey2

    count0 = jnp.zeros((NCLS, 1), jnp.int32)
    count, _ = lax.fori_loop(0, MAXP, body, (count0, maxkey0), unroll=4)

    flags = count < MAXP

    @pl.when(jnp.any(flags))
    def _():
        sfull_ref[:, :] = jnp.where(flags, scores_in_ref[:, :], 0.0)
        flag_l = jnp.broadcast_to(flags, (NCLS, PADP))
        out_ref[0, :, :] = jnp.where(flag_l, 0.0, out_ref[0, :, :])
        out_ref[1, :, :] = jnp.where(flag_l, 0.0, out_ref[1, :, :])
        out_ref[2, :, :] = jnp.where(flag_l, 0.0, out_ref[2, :, :])
        out_ref[3, :, :] = jnp.where(flag_l, 0.0, out_ref[3, :, :])
        out_ref[4, :, :] = jnp.where(flag_l, 0.0, out_ref[4, :, :])

        iota_n = lax.broadcasted_iota(jnp.int32, (NCLS, N), 1)

        def step(i, _):
            scores = sfull_ref[:, :]
            xmin = boxes_ref[0:1, :]
            ymin = boxes_ref[1:2, :]
            xmax = boxes_ref[2:3, :]
            ymax = boxes_ref[3:4, :]

            m = jnp.max(scores, axis=1, keepdims=True)
            idx = jnp.min(jnp.where(scores == m, iota_n, N), axis=1,
                          keepdims=True)
            onehot = (iota_n == idx).astype(jnp.float32)
            b_xmin = jnp.sum(onehot * xmin, axis=1, keepdims=True)
            b_ymin = jnp.sum(onehot * ymin, axis=1, keepdims=True)
            b_xmax = jnp.sum(onehot * xmax, axis=1, keepdims=True)
            b_ymax = jnp.sum(onehot * ymax, axis=1, keepdims=True)
            valid = m > 0.0

            ixmin = jnp.maximum(b_xmin, xmin)
            iymin = jnp.maximum(b_ymin, ymin)
            ixmax = jnp.minimum(b_xmax, xmax)
            iymax = jnp.minimum(b_ymax, ymax)
            iw = jnp.maximum(ixmax - ixmin, 0.0)
            ih = jnp.maximum(iymax - iymin, 0.0)
            inter = iw * ih
            area1 = (jnp.maximum(b_xmax - b_xmin, 0.0)
                     * jnp.maximum(b_ymax - b_ymin, 0.0))
            area2 = (jnp.maximum(xmax - xmin, 0.0)
                     * jnp.maximum(ymax - ymin, 0.0))
            union = area1 + area2 - inter
            iou = inter / jnp.maximum(union, 1e-8)
            sfull_ref[:, :] = jnp.where(iou >= NMST, 0.0, scores)

            sel = (lane == i) & valid
            out_ref[0, :, :] = jnp.where(sel, b_xmin, out_ref[0, :, :])
            out_ref[1, :, :] = jnp.where(sel, b_ymin, out_ref[1, :, :])
            out_ref[2, :, :] = jnp.where(sel, b_xmax, out_ref[2, :, :])
            out_ref[3, :, :] = jnp.where(sel, b_ymax, out_ref[3, :, :])
            out_ref[4, :, :] = jnp.where(sel, m, out_ref[4, :, :])
            return 0

        lax.fori_loop(0, MAXP, step, 0)


def kernel(arm_cls, arm_loc, odm_cls, odm_loc, anchors):
    arm_bg = arm_cls[0, :, 0][None, :]
    arm_loc_t = arm_loc[0].T
    odm_cls_t = odm_cls[0].T[1:NCLS + 1]
    odm_loc_t = odm_loc[0].T
    anchors_t = anchors.T

    scores, boxes8 = pl.pallas_call(
        _decode_body,
        out_shape=(
            jax.ShapeDtypeStruct((NCLS, N), jnp.float32),
            jax.ShapeDtypeStruct((8, N), jnp.float32),
        ),
    )(arm_bg, arm_loc_t, odm_cls_t, odm_loc_t, anchors_t)

    cs, cx, cy, cxx, cyy = _sc_compact(scores, boxes8)

    out = pl.pallas_call(
        _nms_body,
        out_shape=jax.ShapeDtypeStruct((5, NCLS, PADP), jnp.float32),
        scratch_shapes=[
            pltpu.VMEM((NCLS, K), jnp.int32),
            pltpu.VMEM((NCLS, K), jnp.float32),
            pltpu.VMEM((NCLS, N), jnp.float32),
        ],
    )(cs, cx, cy, cxx, cyy, scores, boxes8)

    return jnp.transpose(out[:, :, :MAXP], (1, 2, 0))[None]
